# baseline (speedup 1.0000x reference)
import functools
import math

import jax
import jax.numpy as jnp
import numpy as np
from jax import lax
from jax.experimental import pallas as pl
from jax.experimental.pallas import tpu as pltpu
from jax.experimental.pallas import tpu_sc as plsc

N = 10000
E = 320000
IN_DIM = 128
OUT_DIM = 128
N_HEADS = 4
HEAD_DIM = OUT_DIM // N_HEADS
EDGE_DIM = 16
INV_SCALE = 1.0 / math.sqrt(HEAD_DIM)

NC = 2
NS = 16
NW = NC * NS
EW = E // NW
C = 40
CP = 48
CQ = C + 16
PKW = 224
NG = EW // C
NP = 10240
NPD = NP // 16
NPX = NP + NPD
RPT = NPX // NS

DEN_W = 8


def _proj_body(x_ref, wq_ref, wkv_ref, xq_ref, xkv_ref):
    x = x_ref[...]
    dn = (((1,), (1,)), ((), ()))
    xq_ref[...] = lax.dot_general(x, wq_ref[...], dn,
                                  preferred_element_type=jnp.float32)
    xkv_ref[...] = lax.dot_general(x, wkv_ref[...], dn,
                                   preferred_element_type=jnp.float32)


def _project(x, wq, wkv):
    bn = 2000
    grid = N // bn
    return pl.pallas_call(
        _proj_body,
        grid=(grid,),
        in_specs=[
            pl.BlockSpec((bn, IN_DIM), lambda i: (i, 0)),
            pl.BlockSpec((OUT_DIM, IN_DIM), lambda i: (0, 0)),
            pl.BlockSpec((2 * OUT_DIM, IN_DIM), lambda i: (0, 0)),
        ],
        out_specs=[
            pl.BlockSpec((bn, OUT_DIM), lambda i: (i, 0)),
            pl.BlockSpec((bn, 2 * OUT_DIM), lambda i: (i, 0)),
        ],
        out_shape=[
            jax.ShapeDtypeStruct((N, OUT_DIM), jnp.float32),
            jax.ShapeDtypeStruct((N, 2 * OUT_DIM), jnp.float32),
        ],
    )(x, wq, wkv)


def _bias_body(ea_ref, w1_ref, b1_ref, w2_ref, b2_ref, out_ref):
    dn = (((1,), (1,)), ((), ()))
    z = lax.dot_general(ea_ref[...], w1_ref[...], dn,
                        preferred_element_type=jnp.float32) + b1_ref[...]
    h = z * jax.nn.sigmoid(z)
    out_ref[...] = lax.dot_general(h, w2_ref[...], dn,
                                   preferred_element_type=jnp.float32) + b2_ref[...]


def _edge_bias(edge_attr, w1, b1, w2, b2):
    be = 4000
    grid = E // be
    return pl.pallas_call(
        _bias_body,
        grid=(grid,),
        in_specs=[
            pl.BlockSpec((be, EDGE_DIM), lambda i: (i, 0)),
            pl.BlockSpec((OUT_DIM, EDGE_DIM), lambda i: (0, 0)),
            pl.BlockSpec((1, OUT_DIM), lambda i: (0, 0)),
            pl.BlockSpec((N_HEADS, OUT_DIM), lambda i: (0, 0)),
            pl.BlockSpec((1, N_HEADS), lambda i: (0, 0)),
        ],
        out_specs=pl.BlockSpec((be, N_HEADS), lambda i: (i, 0)),
        out_shape=jax.ShapeDtypeStruct((E, N_HEADS), jnp.float32),
    )(edge_attr, w1, b1, w2, b2)


_GDN = lax.GatherDimensionNumbers(offset_dims=(), collapsed_slice_dims=(0,),
                                  start_index_map=(0,))


def _permute(vec, idx):
    return lax.gather(vec, idx[:, None], _GDN, (1,),
                      mode=lax.GatherScatterMode.PROMISE_IN_BOUNDS)


def _sc_body(xq_hbm, xkv_hbm, pk_hbm, dst_hbm, zeros_hbm,
             out_hbm,
             accum, didxp0, didxp1, didxs0, didxs1, didx20, didx21,
             pkbuf0, pkbuf1,
             qbuf0, qbuf1, kvbuf0, kvbuf1, rowbuf, rowbuf2, gsem, ssem):
    didxp = (didxp0, didxp1)
    didxs = (didxs0, didxs1)
    didx2 = (didx20, didx21)
    pkbuf = (pkbuf0, pkbuf1)
    qbuf = (qbuf0, qbuf1)
    kvbuf = (kvbuf0, kvbuf1)
    c = lax.axis_index("c")
    s = lax.axis_index("s")
    wid = s * NC + c

    zvec = jnp.zeros((16,), jnp.float32)

    pltpu.sync_copy(zeros_hbm.at[pl.ds(s * RPT, RPT)],
                    accum.at[pl.ds(s * RPT, RPT)])
    for b in range(2):
        didxp[b][pl.ds(C, 16)] = jnp.zeros((16,), jnp.int32)
    plsc.subcore_barrier()

    lane = lax.iota(jnp.int32, 16)
    lane4 = lane * 0
    m8 = lane < 8

    def load_idx(b, g):
        pltpu.sync_copy(pk_hbm.at[wid * NG + g], pkbuf[b])
        pltpu.sync_copy(dst_hbm.at[pl.ds(wid * EW + g * C, C)],
                        didxp[b].at[pl.ds(0, C)])

    def start_gather(b):
        pltpu.async_copy(xq_hbm.at[didxp[b].at[pl.ds(0, C)]], qbuf[b], gsem)
        pltpu.async_copy(xkv_hbm.at[pkbuf[b].at[pl.ds(0, C)]], kvbuf[b], gsem)

    def drain_gather(b):
        pltpu.make_async_copy(xq_hbm.at[didxp[b].at[pl.ds(0, C)]], qbuf[b],
                              gsem).wait()
        pltpu.make_async_copy(xkv_hbm.at[pkbuf[b].at[pl.ds(0, C)]], kvbuf[b],
                              gsem).wait()

    def start_scatter(b):
        pltpu.async_copy(rowbuf, accum.at[didxs[b]], ssem, add=True)
        pltpu.async_copy(rowbuf2, accum.at[didx2[b]], ssem, add=True)

    def drain_scatter(b):
        pltpu.make_async_copy(rowbuf, accum.at[didxs[b]], ssem).wait()
        pltpu.make_async_copy(rowbuf2, accum.at[didx2[b]], ssem).wait()

    def compute(b):
        for j in range(CP // 16):
            dv = didxp[b][pl.ds(16 * j, 16)]
            d2 = lax.shift_right_logical(dv, 4) + NP
            if 16 * (j + 1) <= C:
                didxs[b][pl.ds(16 * j, 16)] = dv
                didx2[b][pl.ds(16 * j, 16)] = d2
            else:
                plsc.store_scatter(didxs[b], [lane + 16 * j], dv, mask=m8)
                plsc.store_scatter(didx2[b], [lane + 16 * j], d2, mask=m8)

        @plsc.parallel_loop(0, C, 1, unroll=4)
        def edge(i):
            bvec_i = pkbuf[b][pl.ds(C + 8 + i * N_HEADS, 16)]
            bvec = plsc.bitcast(bvec_i, jnp.float32)
            ex = []
            for h in range(N_HEADS):
                q0 = qbuf[b][i, pl.ds(32 * h, 16)]
                q1 = qbuf[b][i, pl.ds(32 * h + 16, 16)]
                k0 = kvbuf[b][i, pl.ds(32 * h, 16)]
                k1 = kvbuf[b][i, pl.ds(32 * h + 16, 16)]
                r = q0 * k0 + q1 * k1
                for sh in (8, 4, 2, 1):
                    r = r + _permute(r, lane ^ sh)
                b_h = _permute(bvec, lane4 + h)
                e_h = jnp.exp(r * INV_SCALE + b_h)
                ex.append(e_h)
                v0 = kvbuf[b][i, pl.ds(128 + 32 * h, 16)]
                v1 = kvbuf[b][i, pl.ds(128 + 32 * h + 16, 16)]
                rowbuf[i, pl.ds(32 * h, 16)] = v0 * e_h
                rowbuf[i, pl.ds(32 * h + 16, 16)] = v1 * e_h
            den = jnp.where(
                lane == 0, ex[0],
                jnp.where(lane == 1, ex[1],
                          jnp.where(lane == 2, ex[2],
                                    jnp.where(lane == 3, ex[3], 1.0))))
            dvec = didxp[b][pl.ds(i, 16)]
            pos = dvec[0] & 15
            sh8 = (pos & 1) * 8
            perm = (lane - sh8) & 15
            den_m = jnp.where(perm < 5, _permute(den, perm), 0.0)
            grp = lax.shift_right_logical(pos, 1)
            for k in range(8):
                rowbuf2[i, pl.ds(16 * k, 16)] = jnp.where(grp == k, den_m,
                                                          zvec)

    load_idx(0, 0)
    start_gather(0)

    def outer(t, carry):
        for b in range(2):
            g = 2 * t + b
            nb = 1 - b

            @pl.when(g > 0)
            def _():
                drain_scatter(nb)

            @pl.when(g + 1 < NG)
            def _():
                load_idx(nb, g + 1)
                start_gather(nb)

            drain_gather(b)
            compute(b)
            start_scatter(b)
        return carry

    lax.fori_loop(0, NG // 2, outer, 0)
    drain_scatter(1)

    plsc.subcore_barrier()
    pltpu.sync_copy(accum.at[pl.ds(s * RPT, RPT)],
                    out_hbm.at[c, pl.ds(s * RPT, RPT)])


def _sc_aggregate(xq, xkv, pk, dst, zeros):
    mesh = plsc.VectorSubcoreMesh(core_axis_name="c", subcore_axis_name="s")
    fn = pl.kernel(
        _sc_body,
        out_type=jax.ShapeDtypeStruct((NC, NPX, OUT_DIM), jnp.float32),
        mesh=mesh,
        scratch_types=[
            pltpu.VMEM_SHARED((NPX, OUT_DIM), jnp.float32),
            pltpu.VMEM((CQ,), jnp.int32),
            pltpu.VMEM((CQ,), jnp.int32),
            pltpu.VMEM((C,), jnp.int32),
            pltpu.VMEM((C,), jnp.int32),
            pltpu.VMEM((C,), jnp.int32),
            pltpu.VMEM((C,), jnp.int32),
            pltpu.VMEM((PKW,), jnp.int32),
            pltpu.VMEM((PKW,), jnp.int32),
            pltpu.VMEM((C, OUT_DIM), jnp.float32),
            pltpu.VMEM((C, OUT_DIM), jnp.float32),
            pltpu.VMEM((C, 2 * OUT_DIM), jnp.float32),
            pltpu.VMEM((C, 2 * OUT_DIM), jnp.float32),
            pltpu.VMEM((C, OUT_DIM), jnp.float32),
            pltpu.VMEM((C, OUT_DIM), jnp.float32),
            pltpu.SemaphoreType.DMA,
            pltpu.SemaphoreType.DMA,
        ],
        compiler_params=pltpu.CompilerParams(needs_layout_passes=False),
    )
    return fn(xq, xkv, pk, dst, zeros)


_EXPAND = np.kron(np.eye(N_HEADS, dtype=np.float32),
                  np.ones((1, HEAD_DIM), dtype=np.float32))


def _final_body(num_ref, den_ref, x_ref, wo_ref, wob_ref, g_ref, b_ref,
                exp_ref, out_ref):
    num = num_ref[0] + num_ref[1]
    dacc = den_ref[0] + den_ref[1]
    den4 = dacc[:, :N_HEADS]
    deg = dacc[:, N_HEADS:N_HEADS + 1]
    den = jnp.dot(den4, exp_ref[...], preferred_element_type=jnp.float32)
    aggr = num / (den + 1e-16)
    dn = (((1,), (1,)), ((), ()))
    msg = (lax.dot_general(aggr, wo_ref[...], dn,
                           preferred_element_type=jnp.float32)
           + deg * wob_ref[...])
    ge = 0.5 * msg * (1.0 + lax.erf(msg * (1.0 / math.sqrt(2.0))))
    y = x_ref[...] + ge
    mu = jnp.mean(y, axis=-1, keepdims=True)
    var = jnp.mean((y - mu) ** 2, axis=-1, keepdims=True)
    out_ref[...] = (y - mu) * lax.rsqrt(var + 1e-5) * g_ref[...] + b_ref[...]


def _finalize(num, den, x, wo, wob, ln_g, ln_b):
    bn = 2000
    grid = N // bn
    return pl.pallas_call(
        _final_body,
        grid=(grid,),
        in_specs=[
            pl.BlockSpec((NC, bn, OUT_DIM), lambda i: (0, i, 0)),
            pl.BlockSpec((NC, bn, DEN_W), lambda i: (0, i, 0)),
            pl.BlockSpec((bn, OUT_DIM), lambda i: (i, 0)),
            pl.BlockSpec((OUT_DIM, OUT_DIM), lambda i: (0, 0)),
            pl.BlockSpec((1, OUT_DIM), lambda i: (0, 0)),
            pl.BlockSpec((1, OUT_DIM), lambda i: (0, 0)),
            pl.BlockSpec((1, OUT_DIM), lambda i: (0, 0)),
            pl.BlockSpec((N_HEADS, OUT_DIM), lambda i: (0, 0)),
        ],
        out_specs=pl.BlockSpec((bn, OUT_DIM), lambda i: (i, 0)),
        out_shape=jax.ShapeDtypeStruct((N, OUT_DIM), jnp.float32),
    )(num, den, x, wo, wob, ln_g, ln_b, jnp.asarray(_EXPAND))


def kernel(x, edge_index, edge_attr, Wq, Wk, Wv, Ep1_w, Ep1_b, Ep2_w, Ep2_b,
           Wo_w, Wo_b, ln_g, ln_b):
    wkv = jnp.concatenate([Wk, Wv], axis=0)
    xq, xkv = _project(x, Wq, wkv)
    bias = _edge_bias(edge_attr, Ep1_w, Ep1_b.reshape(1, -1),
                      Ep2_w, Ep2_b.reshape(1, -1))
    src = edge_index[0]
    dst = edge_index[1]
    nchunks = NW * NG
    bias_i = lax.bitcast_convert_type(bias, jnp.int32)
    pk = jnp.concatenate(
        [src.reshape(nchunks, C),
         jnp.zeros((nchunks, 8), jnp.int32),
         bias_i.reshape(nchunks, C * N_HEADS),
         jnp.zeros((nchunks, PKW - CQ - C * N_HEADS + 8), jnp.int32)],
        axis=1)
    zeros = jnp.zeros((NPX, OUT_DIM), jnp.float32)
    parts = _sc_aggregate(xq, xkv, pk, dst, zeros)
    num = parts[:, :NP, :]
    den = parts[:, NP:, :].reshape(NC, NP, DEN_W)
    return _finalize(num, den, x, Wo_w,
                     Wo_b.reshape(1, -1), ln_g.reshape(1, -1),
                     ln_b.reshape(1, -1))


# speedup vs baseline: 1.1719x; 1.1719x over previous
import functools
import math

import jax
import jax.numpy as jnp
import numpy as np
from jax import lax
from jax.experimental import pallas as pl
from jax.experimental.pallas import tpu as pltpu
from jax.experimental.pallas import tpu_sc as plsc

N = 10000
E = 320000
IN_DIM = 128
OUT_DIM = 128
N_HEADS = 4
HEAD_DIM = OUT_DIM // N_HEADS
EDGE_DIM = 16
INV_SCALE = 1.0 / math.sqrt(HEAD_DIM)

NC = 2
NS = 16
NW = NC * NS
EW = E // NW
C = 40
CP = 48
CQ = C + 16
PKW = 224
NG = EW // C
NP = 10240
NPD = NP // 16
NPX = NP + NPD
RPT = NPX // NS

DEN_W = 8


def _proj_body(x_ref, wq_ref, wkv_ref, xq_ref, xkv_ref):
    x = x_ref[...]
    dn = (((1,), (1,)), ((), ()))
    xq_ref[...] = lax.dot_general(x, wq_ref[...], dn,
                                  preferred_element_type=jnp.float32)
    xkv_ref[...] = lax.dot_general(x, wkv_ref[...], dn,
                                   preferred_element_type=jnp.float32)


def _project(x, wq, wkv):
    bn = 2000
    grid = N // bn
    return pl.pallas_call(
        _proj_body,
        grid=(grid,),
        in_specs=[
            pl.BlockSpec((bn, IN_DIM), lambda i: (i, 0)),
            pl.BlockSpec((OUT_DIM, IN_DIM), lambda i: (0, 0)),
            pl.BlockSpec((2 * OUT_DIM, IN_DIM), lambda i: (0, 0)),
        ],
        out_specs=[
            pl.BlockSpec((bn, OUT_DIM), lambda i: (i, 0)),
            pl.BlockSpec((bn, 2 * OUT_DIM), lambda i: (i, 0)),
        ],
        out_shape=[
            jax.ShapeDtypeStruct((N, OUT_DIM), jnp.float32),
            jax.ShapeDtypeStruct((N, 2 * OUT_DIM), jnp.float32),
        ],
    )(x, wq, wkv)


def _bias_body(ea_ref, w1_ref, b1_ref, w2_ref, b2_ref, out_ref):
    dn = (((1,), (1,)), ((), ()))
    z = lax.dot_general(ea_ref[...], w1_ref[...], dn,
                        preferred_element_type=jnp.float32) + b1_ref[...]
    h = z * jax.nn.sigmoid(z)
    out_ref[...] = lax.dot_general(h, w2_ref[...], dn,
                                   preferred_element_type=jnp.float32) + b2_ref[...]


def _edge_bias(edge_attr, w1, b1, w2, b2):
    be = 4000
    grid = E // be
    return pl.pallas_call(
        _bias_body,
        grid=(grid,),
        in_specs=[
            pl.BlockSpec((be, EDGE_DIM), lambda i: (i, 0)),
            pl.BlockSpec((OUT_DIM, EDGE_DIM), lambda i: (0, 0)),
            pl.BlockSpec((1, OUT_DIM), lambda i: (0, 0)),
            pl.BlockSpec((N_HEADS, OUT_DIM), lambda i: (0, 0)),
            pl.BlockSpec((1, N_HEADS), lambda i: (0, 0)),
        ],
        out_specs=pl.BlockSpec((be, N_HEADS), lambda i: (i, 0)),
        out_shape=jax.ShapeDtypeStruct((E, N_HEADS), jnp.float32),
    )(edge_attr, w1, b1, w2, b2)


_GDN = lax.GatherDimensionNumbers(offset_dims=(), collapsed_slice_dims=(0,),
                                  start_index_map=(0,))


def _permute(vec, idx):
    return lax.gather(vec, idx[:, None], _GDN, (1,),
                      mode=lax.GatherScatterMode.PROMISE_IN_BOUNDS)


def _sc_body(xq_hbm, xkv_hbm, pk_hbm, dst_hbm, zeros_hbm,
             out_hbm,
             accum, didxp0, didxp1, didxs0, didxs1, didx20, didx21,
             pkbuf0, pkbuf1,
             qbuf0, qbuf1, kvbuf0, kvbuf1, rowbuf, rowbuf2, gsem, ssem):
    didxp = (didxp0, didxp1)
    didxs = (didxs0, didxs1)
    didx2 = (didx20, didx21)
    pkbuf = (pkbuf0, pkbuf1)
    qbuf = (qbuf0, qbuf1)
    kvbuf = (kvbuf0, kvbuf1)
    c = lax.axis_index("c")
    s = lax.axis_index("s")
    wid = s * NC + c

    zvec = jnp.zeros((16,), jnp.float32)

    pltpu.sync_copy(zeros_hbm.at[pl.ds(s * RPT, RPT)],
                    accum.at[pl.ds(s * RPT, RPT)])
    for b in range(2):
        didxp[b][pl.ds(C, 16)] = jnp.zeros((16,), jnp.int32)
    plsc.subcore_barrier()

    lane = lax.iota(jnp.int32, 16)
    lane4 = lane * 0
    m8 = lane < 8

    def load_idx(b, g):
        pltpu.sync_copy(pk_hbm.at[wid * NG + g], pkbuf[b])
        pltpu.sync_copy(dst_hbm.at[pl.ds(wid * EW + g * C, C)],
                        didxp[b].at[pl.ds(0, C)])

    def start_gather(b):
        pltpu.async_copy(xq_hbm.at[didxp[b].at[pl.ds(0, C)]], qbuf[b], gsem)
        pltpu.async_copy(xkv_hbm.at[pkbuf[b].at[pl.ds(0, C)]], kvbuf[b], gsem)

    def drain_gather(b):
        pltpu.make_async_copy(xq_hbm.at[didxp[b].at[pl.ds(0, C)]], qbuf[b],
                              gsem).wait()
        pltpu.make_async_copy(xkv_hbm.at[pkbuf[b].at[pl.ds(0, C)]], kvbuf[b],
                              gsem).wait()

    def start_scatter(b):
        pltpu.async_copy(rowbuf, accum.at[didxs[b]], ssem, add=True)
        pltpu.async_copy(rowbuf2, accum.at[didx2[b]], ssem, add=True)

    def drain_scatter(b):
        pltpu.make_async_copy(rowbuf, accum.at[didxs[b]], ssem).wait()
        pltpu.make_async_copy(rowbuf2, accum.at[didx2[b]], ssem).wait()

    def compute(b):
        for j in range(CP // 16):
            dv = didxp[b][pl.ds(16 * j, 16)]
            d2 = lax.shift_right_logical(dv, 4) + NP
            if 16 * (j + 1) <= C:
                didxs[b][pl.ds(16 * j, 16)] = dv
                didx2[b][pl.ds(16 * j, 16)] = d2
            else:
                plsc.store_scatter(didxs[b], [lane + 16 * j], dv, mask=m8)
                plsc.store_scatter(didx2[b], [lane + 16 * j], d2, mask=m8)

        @plsc.parallel_loop(0, C, 1, unroll=2)
        def edge(i):
            bvec_i = pkbuf[b][pl.ds(C + 8 + i * N_HEADS, 16)]
            bvec = plsc.bitcast(bvec_i, jnp.float32)
            ex = []
            for h in range(N_HEADS):
                q0 = qbuf[b][i, pl.ds(32 * h, 16)]
                q1 = qbuf[b][i, pl.ds(32 * h + 16, 16)]
                k0 = kvbuf[b][i, pl.ds(32 * h, 16)]
                k1 = kvbuf[b][i, pl.ds(32 * h + 16, 16)]
                r = q0 * k0 + q1 * k1
                for sh in (8, 4, 2, 1):
                    r = r + _permute(r, lane ^ sh)
                b_h = _permute(bvec, lane4 + h)
                e_h = jnp.exp(r * INV_SCALE + b_h)
                ex.append(e_h)
                v0 = kvbuf[b][i, pl.ds(128 + 32 * h, 16)]
                v1 = kvbuf[b][i, pl.ds(128 + 32 * h + 16, 16)]
                rowbuf[i, pl.ds(32 * h, 16)] = v0 * e_h
                rowbuf[i, pl.ds(32 * h + 16, 16)] = v1 * e_h
            den = jnp.where(
                lane == 0, ex[0],
                jnp.where(lane == 1, ex[1],
                          jnp.where(lane == 2, ex[2],
                                    jnp.where(lane == 3, ex[3], 1.0))))
            dvec = didxp[b][pl.ds(i, 16)]
            pos = dvec[0] & 15
            sh8 = (pos & 1) * 8
            perm = (lane - sh8) & 15
            den_m = jnp.where(perm < 5, _permute(den, perm), 0.0)
            grp = lax.shift_right_logical(pos, 1)
            for k in range(8):
                rowbuf2[i, pl.ds(16 * k, 16)] = jnp.where(grp == k, den_m,
                                                          zvec)

    load_idx(0, 0)
    start_gather(0)

    def outer(t, carry):
        for b in range(2):
            g = 2 * t + b
            nb = 1 - b

            @pl.when(g > 0)
            def _():
                drain_scatter(nb)

            @pl.when(g + 1 < NG)
            def _():
                load_idx(nb, g + 1)
                start_gather(nb)

            drain_gather(b)
            compute(b)
            start_scatter(b)
        return carry

    lax.fori_loop(0, NG // 2, outer, 0)
    drain_scatter(1)

    plsc.subcore_barrier()
    pltpu.sync_copy(accum.at[pl.ds(s * RPT, RPT)],
                    out_hbm.at[c, pl.ds(s * RPT, RPT)])


def _sc_aggregate(xq, xkv, pk, dst, zeros):
    mesh = plsc.VectorSubcoreMesh(core_axis_name="c", subcore_axis_name="s")
    fn = pl.kernel(
        _sc_body,
        out_type=jax.ShapeDtypeStruct((NC, NPX, OUT_DIM), jnp.float32),
        mesh=mesh,
        scratch_types=[
            pltpu.VMEM_SHARED((NPX, OUT_DIM), jnp.float32),
            pltpu.VMEM((CQ,), jnp.int32),
            pltpu.VMEM((CQ,), jnp.int32),
            pltpu.VMEM((C,), jnp.int32),
            pltpu.VMEM((C,), jnp.int32),
            pltpu.VMEM((C,), jnp.int32),
            pltpu.VMEM((C,), jnp.int32),
            pltpu.VMEM((PKW,), jnp.int32),
            pltpu.VMEM((PKW,), jnp.int32),
            pltpu.VMEM((C, OUT_DIM), jnp.float32),
            pltpu.VMEM((C, OUT_DIM), jnp.float32),
            pltpu.VMEM((C, 2 * OUT_DIM), jnp.float32),
            pltpu.VMEM((C, 2 * OUT_DIM), jnp.float32),
            pltpu.VMEM((C, OUT_DIM), jnp.float32),
            pltpu.VMEM((C, OUT_DIM), jnp.float32),
            pltpu.SemaphoreType.DMA,
            pltpu.SemaphoreType.DMA,
        ],
        compiler_params=pltpu.CompilerParams(needs_layout_passes=False),
    )
    return fn(xq, xkv, pk, dst, zeros)


_EXPAND = np.kron(np.eye(N_HEADS, dtype=np.float32),
                  np.ones((1, HEAD_DIM), dtype=np.float32))


def _final_body(num_ref, den_ref, x_ref, wo_ref, wob_ref, g_ref, b_ref,
                exp_ref, out_ref):
    num = num_ref[0] + num_ref[1]
    dacc = den_ref[0] + den_ref[1]
    den4 = dacc[:, :N_HEADS]
    deg = dacc[:, N_HEADS:N_HEADS + 1]
    den = jnp.dot(den4, exp_ref[...], preferred_element_type=jnp.float32)
    aggr = num / (den + 1e-16)
    dn = (((1,), (1,)), ((), ()))
    msg = (lax.dot_general(aggr, wo_ref[...], dn,
                           preferred_element_type=jnp.float32)
           + deg * wob_ref[...])
    ge = 0.5 * msg * (1.0 + lax.erf(msg * (1.0 / math.sqrt(2.0))))
    y = x_ref[...] + ge
    mu = jnp.mean(y, axis=-1, keepdims=True)
    var = jnp.mean((y - mu) ** 2, axis=-1, keepdims=True)
    out_ref[...] = (y - mu) * lax.rsqrt(var + 1e-5) * g_ref[...] + b_ref[...]


def _finalize(num, den, x, wo, wob, ln_g, ln_b):
    bn = 2000
    grid = N // bn
    return pl.pallas_call(
        _final_body,
        grid=(grid,),
        in_specs=[
            pl.BlockSpec((NC, bn, OUT_DIM), lambda i: (0, i, 0)),
            pl.BlockSpec((NC, bn, DEN_W), lambda i: (0, i, 0)),
            pl.BlockSpec((bn, OUT_DIM), lambda i: (i, 0)),
            pl.BlockSpec((OUT_DIM, OUT_DIM), lambda i: (0, 0)),
            pl.BlockSpec((1, OUT_DIM), lambda i: (0, 0)),
            pl.BlockSpec((1, OUT_DIM), lambda i: (0, 0)),
            pl.BlockSpec((1, OUT_DIM), lambda i: (0, 0)),
            pl.BlockSpec((N_HEADS, OUT_DIM), lambda i: (0, 0)),
        ],
        out_specs=pl.BlockSpec((bn, OUT_DIM), lambda i: (i, 0)),
        out_shape=jax.ShapeDtypeStruct((N, OUT_DIM), jnp.float32),
    )(num, den, x, wo, wob, ln_g, ln_b, jnp.asarray(_EXPAND))


def kernel(x, edge_index, edge_attr, Wq, Wk, Wv, Ep1_w, Ep1_b, Ep2_w, Ep2_b,
           Wo_w, Wo_b, ln_g, ln_b):
    wkv = jnp.concatenate([Wk, Wv], axis=0)
    xq, xkv = _project(x, Wq, wkv)
    bias = _edge_bias(edge_attr, Ep1_w, Ep1_b.reshape(1, -1),
                      Ep2_w, Ep2_b.reshape(1, -1))
    src = edge_index[0]
    dst = edge_index[1]
    nchunks = NW * NG
    bias_i = lax.bitcast_convert_type(bias, jnp.int32)
    pk = jnp.concatenate(
        [src.reshape(nchunks, C),
         jnp.zeros((nchunks, 8), jnp.int32),
         bias_i.reshape(nchunks, C * N_HEADS),
         jnp.zeros((nchunks, PKW - CQ - C * N_HEADS + 8), jnp.int32)],
        axis=1)
    zeros = jnp.zeros((NPX, OUT_DIM), jnp.float32)
    parts = _sc_aggregate(xq, xkv, pk, dst, zeros)
    num = parts[:, :NP, :]
    den = parts[:, NP:, :].reshape(NC, NP, DEN_W)
    return _finalize(num, den, x, Wo_w,
                     Wo_b.reshape(1, -1), ln_g.reshape(1, -1),
                     ln_b.reshape(1, -1))


# speedup vs baseline: 1.1872x; 1.0131x over previous
import functools
import math

import jax
import jax.numpy as jnp
import numpy as np
from jax import lax
from jax.experimental import pallas as pl
from jax.experimental.pallas import tpu as pltpu
from jax.experimental.pallas import tpu_sc as plsc

N = 10000
E = 320000
IN_DIM = 128
OUT_DIM = 128
N_HEADS = 4
HEAD_DIM = OUT_DIM // N_HEADS
EDGE_DIM = 16
INV_SCALE = 1.0 / math.sqrt(HEAD_DIM)

NC = 2
NS = 16
NW = NC * NS
EW = E // NW
C = 40
CP = 48
CQ = C + 16
PKW = 224
NG = EW // C
NP = 10240
NPD = NP // 16
NPX = NP + NPD
RPT = NPX // NS

DEN_W = 8


def _proj_body(x_ref, wq_ref, wkv_ref, xq_ref, xkv_ref):
    x = x_ref[...]
    dn = (((1,), (1,)), ((), ()))
    xq_ref[...] = lax.dot_general(x, wq_ref[...], dn,
                                  preferred_element_type=jnp.float32)
    xkv_ref[...] = lax.dot_general(x, wkv_ref[...], dn,
                                   preferred_element_type=jnp.float32)


def _project(x, wq, wkv):
    bn = 2000
    grid = N // bn
    return pl.pallas_call(
        _proj_body,
        grid=(grid,),
        in_specs=[
            pl.BlockSpec((bn, IN_DIM), lambda i: (i, 0)),
            pl.BlockSpec((OUT_DIM, IN_DIM), lambda i: (0, 0)),
            pl.BlockSpec((2 * OUT_DIM, IN_DIM), lambda i: (0, 0)),
        ],
        out_specs=[
            pl.BlockSpec((bn, OUT_DIM), lambda i: (i, 0)),
            pl.BlockSpec((bn, 2 * OUT_DIM), lambda i: (i, 0)),
        ],
        out_shape=[
            jax.ShapeDtypeStruct((N, OUT_DIM), jnp.float32),
            jax.ShapeDtypeStruct((N, 2 * OUT_DIM), jnp.float32),
        ],
    )(x, wq, wkv)


def _bias_body(ea_ref, w1_ref, b1_ref, w2_ref, b2_ref, out_ref):
    dn = (((1,), (1,)), ((), ()))
    z = lax.dot_general(ea_ref[...], w1_ref[...], dn,
                        preferred_element_type=jnp.float32) + b1_ref[...]
    h = z * jax.nn.sigmoid(z)
    out_ref[...] = lax.dot_general(h, w2_ref[...], dn,
                                   preferred_element_type=jnp.float32) + b2_ref[...]


def _edge_bias(edge_attr, w1, b1, w2, b2):
    be = 4000
    grid = E // be
    return pl.pallas_call(
        _bias_body,
        grid=(grid,),
        in_specs=[
            pl.BlockSpec((be, EDGE_DIM), lambda i: (i, 0)),
            pl.BlockSpec((OUT_DIM, EDGE_DIM), lambda i: (0, 0)),
            pl.BlockSpec((1, OUT_DIM), lambda i: (0, 0)),
            pl.BlockSpec((N_HEADS, OUT_DIM), lambda i: (0, 0)),
            pl.BlockSpec((1, N_HEADS), lambda i: (0, 0)),
        ],
        out_specs=pl.BlockSpec((be, N_HEADS), lambda i: (i, 0)),
        out_shape=jax.ShapeDtypeStruct((E, N_HEADS), jnp.float32),
    )(edge_attr, w1, b1, w2, b2)


_GDN = lax.GatherDimensionNumbers(offset_dims=(), collapsed_slice_dims=(0,),
                                  start_index_map=(0,))


def _permute(vec, idx):
    return lax.gather(vec, idx[:, None], _GDN, (1,),
                      mode=lax.GatherScatterMode.PROMISE_IN_BOUNDS)


def _sc_body(xq_hbm, xkv_hbm, pk_hbm, dst_hbm, zeros_hbm,
             out_hbm,
             accum, didxp0, didxp1, didxs0, didxs1, didx20, didx21,
             pkbuf0, pkbuf1,
             qbuf0, qbuf1, kvbuf0, kvbuf1, rowbuf, rowbuf2, gsem, ssem):
    didxp = (didxp0, didxp1)
    didxs = (didxs0, didxs1)
    didx2 = (didx20, didx21)
    pkbuf = (pkbuf0, pkbuf1)
    qbuf = (qbuf0, qbuf1)
    kvbuf = (kvbuf0, kvbuf1)
    c = lax.axis_index("c")
    s = lax.axis_index("s")
    wid = s * NC + c

    zvec = jnp.zeros((16,), jnp.float32)

    pltpu.sync_copy(zeros_hbm.at[pl.ds(s * RPT, RPT)],
                    accum.at[pl.ds(s * RPT, RPT)])
    for b in range(2):
        didxp[b][pl.ds(C, 16)] = jnp.zeros((16,), jnp.int32)
    plsc.subcore_barrier()

    lane = lax.iota(jnp.int32, 16)
    lane4 = lane * 0
    m8 = lane < 8

    def load_idx(b, g):
        pltpu.sync_copy(pk_hbm.at[wid * NG + g], pkbuf[b])
        pltpu.sync_copy(dst_hbm.at[pl.ds(wid * EW + g * C, C)],
                        didxp[b].at[pl.ds(0, C)])

    def start_gather(b):
        pltpu.async_copy(xq_hbm.at[didxp[b].at[pl.ds(0, C)]], qbuf[b], gsem)
        pltpu.async_copy(xkv_hbm.at[pkbuf[b].at[pl.ds(0, C)]], kvbuf[b], gsem)

    def drain_gather(b):
        pltpu.make_async_copy(xq_hbm.at[didxp[b].at[pl.ds(0, C)]], qbuf[b],
                              gsem).wait()
        pltpu.make_async_copy(xkv_hbm.at[pkbuf[b].at[pl.ds(0, C)]], kvbuf[b],
                              gsem).wait()

    def start_scatter(b):
        pltpu.async_copy(rowbuf, accum.at[didxs[b]], ssem, add=True)
        pltpu.async_copy(rowbuf2, accum.at[didx2[b]], ssem, add=True)

    def drain_scatter(b):
        pltpu.make_async_copy(rowbuf, accum.at[didxs[b]], ssem).wait()
        pltpu.make_async_copy(rowbuf2, accum.at[didx2[b]], ssem).wait()

    def compute(b):
        for j in range(CP // 16):
            dv = didxp[b][pl.ds(16 * j, 16)]
            d2 = lax.shift_right_logical(dv, 4) + NP
            if 16 * (j + 1) <= C:
                didxs[b][pl.ds(16 * j, 16)] = dv
                didx2[b][pl.ds(16 * j, 16)] = d2
            else:
                plsc.store_scatter(didxs[b], [lane + 16 * j], dv, mask=m8)
                plsc.store_scatter(didx2[b], [lane + 16 * j], d2, mask=m8)

        @plsc.parallel_loop(0, C, 1, unroll=2)
        def edge(i):
            bvec_i = pkbuf[b][pl.ds(C + 8 + i * N_HEADS, 16)]
            bvec = plsc.bitcast(bvec_i, jnp.float32)
            ex = []
            for h in range(N_HEADS):
                q0 = qbuf[b][i, pl.ds(32 * h, 16)]
                q1 = qbuf[b][i, pl.ds(32 * h + 16, 16)]
                k0 = kvbuf[b][i, pl.ds(32 * h, 16)]
                k1 = kvbuf[b][i, pl.ds(32 * h + 16, 16)]
                cs = plsc.cumsum(q0 * k0 + q1 * k1)
                logit = cs[15] * INV_SCALE + bvec[h]
                e_h = jnp.exp(jnp.full((16,), logit, jnp.float32))
                ex.append(e_h)
                v0 = kvbuf[b][i, pl.ds(128 + 32 * h, 16)]
                v1 = kvbuf[b][i, pl.ds(128 + 32 * h + 16, 16)]
                rowbuf[i, pl.ds(32 * h, 16)] = v0 * e_h
                rowbuf[i, pl.ds(32 * h + 16, 16)] = v1 * e_h
            den = jnp.where(
                lane == 0, ex[0],
                jnp.where(lane == 1, ex[1],
                          jnp.where(lane == 2, ex[2],
                                    jnp.where(lane == 3, ex[3], 1.0))))
            dvec = didxp[b][pl.ds(i, 16)]
            pos = dvec[0] & 15
            sh8 = (pos & 1) * 8
            perm = (lane - sh8) & 15
            den_m = jnp.where(perm < 5, _permute(den, perm), 0.0)
            grp = lax.shift_right_logical(pos, 1)
            for k in range(8):
                rowbuf2[i, pl.ds(16 * k, 16)] = jnp.where(grp == k, den_m,
                                                          zvec)

    load_idx(0, 0)
    start_gather(0)

    def outer(t, carry):
        for b in range(2):
            g = 2 * t + b
            nb = 1 - b

            @pl.when(g > 0)
            def _():
                drain_scatter(nb)

            @pl.when(g + 1 < NG)
            def _():
                load_idx(nb, g + 1)
                start_gather(nb)

            drain_gather(b)
            compute(b)
            start_scatter(b)
        return carry

    lax.fori_loop(0, NG // 2, outer, 0)
    drain_scatter(1)

    plsc.subcore_barrier()
    pltpu.sync_copy(accum.at[pl.ds(s * RPT, RPT)],
                    out_hbm.at[c, pl.ds(s * RPT, RPT)])


def _sc_aggregate(xq, xkv, pk, dst, zeros):
    mesh = plsc.VectorSubcoreMesh(core_axis_name="c", subcore_axis_name="s")
    fn = pl.kernel(
        _sc_body,
        out_type=jax.ShapeDtypeStruct((NC, NPX, OUT_DIM), jnp.float32),
        mesh=mesh,
        scratch_types=[
            pltpu.VMEM_SHARED((NPX, OUT_DIM), jnp.float32),
            pltpu.VMEM((CQ,), jnp.int32),
            pltpu.VMEM((CQ,), jnp.int32),
            pltpu.VMEM((C,), jnp.int32),
            pltpu.VMEM((C,), jnp.int32),
            pltpu.VMEM((C,), jnp.int32),
            pltpu.VMEM((C,), jnp.int32),
            pltpu.VMEM((PKW,), jnp.int32),
            pltpu.VMEM((PKW,), jnp.int32),
            pltpu.VMEM((C, OUT_DIM), jnp.float32),
            pltpu.VMEM((C, OUT_DIM), jnp.float32),
            pltpu.VMEM((C, 2 * OUT_DIM), jnp.float32),
            pltpu.VMEM((C, 2 * OUT_DIM), jnp.float32),
            pltpu.VMEM((C, OUT_DIM), jnp.float32),
            pltpu.VMEM((C, OUT_DIM), jnp.float32),
            pltpu.SemaphoreType.DMA,
            pltpu.SemaphoreType.DMA,
        ],
        compiler_params=pltpu.CompilerParams(needs_layout_passes=False),
    )
    return fn(xq, xkv, pk, dst, zeros)


_EXPAND = np.kron(np.eye(N_HEADS, dtype=np.float32),
                  np.ones((1, HEAD_DIM), dtype=np.float32))


def _final_body(num_ref, den_ref, x_ref, wo_ref, wob_ref, g_ref, b_ref,
                exp_ref, out_ref):
    num = num_ref[0] + num_ref[1]
    dacc = den_ref[0] + den_ref[1]
    den4 = dacc[:, :N_HEADS]
    deg = dacc[:, N_HEADS:N_HEADS + 1]
    den = jnp.dot(den4, exp_ref[...], preferred_element_type=jnp.float32)
    aggr = num / (den + 1e-16)
    dn = (((1,), (1,)), ((), ()))
    msg = (lax.dot_general(aggr, wo_ref[...], dn,
                           preferred_element_type=jnp.float32)
           + deg * wob_ref[...])
    ge = 0.5 * msg * (1.0 + lax.erf(msg * (1.0 / math.sqrt(2.0))))
    y = x_ref[...] + ge
    mu = jnp.mean(y, axis=-1, keepdims=True)
    var = jnp.mean((y - mu) ** 2, axis=-1, keepdims=True)
    out_ref[...] = (y - mu) * lax.rsqrt(var + 1e-5) * g_ref[...] + b_ref[...]


def _finalize(num, den, x, wo, wob, ln_g, ln_b):
    bn = 2000
    grid = N // bn
    return pl.pallas_call(
        _final_body,
        grid=(grid,),
        in_specs=[
            pl.BlockSpec((NC, bn, OUT_DIM), lambda i: (0, i, 0)),
            pl.BlockSpec((NC, bn, DEN_W), lambda i: (0, i, 0)),
            pl.BlockSpec((bn, OUT_DIM), lambda i: (i, 0)),
            pl.BlockSpec((OUT_DIM, OUT_DIM), lambda i: (0, 0)),
            pl.BlockSpec((1, OUT_DIM), lambda i: (0, 0)),
            pl.BlockSpec((1, OUT_DIM), lambda i: (0, 0)),
            pl.BlockSpec((1, OUT_DIM), lambda i: (0, 0)),
            pl.BlockSpec((N_HEADS, OUT_DIM), lambda i: (0, 0)),
        ],
        out_specs=pl.BlockSpec((bn, OUT_DIM), lambda i: (i, 0)),
        out_shape=jax.ShapeDtypeStruct((N, OUT_DIM), jnp.float32),
    )(num, den, x, wo, wob, ln_g, ln_b, jnp.asarray(_EXPAND))


def kernel(x, edge_index, edge_attr, Wq, Wk, Wv, Ep1_w, Ep1_b, Ep2_w, Ep2_b,
           Wo_w, Wo_b, ln_g, ln_b):
    wkv = jnp.concatenate([Wk, Wv], axis=0)
    xq, xkv = _project(x, Wq, wkv)
    bias = _edge_bias(edge_attr, Ep1_w, Ep1_b.reshape(1, -1),
                      Ep2_w, Ep2_b.reshape(1, -1))
    src = edge_index[0]
    dst = edge_index[1]
    nchunks = NW * NG
    bias_i = lax.bitcast_convert_type(bias, jnp.int32)
    pk = jnp.concatenate(
        [src.reshape(nchunks, C),
         jnp.zeros((nchunks, 8), jnp.int32),
         bias_i.reshape(nchunks, C * N_HEADS),
         jnp.zeros((nchunks, PKW - CQ - C * N_HEADS + 8), jnp.int32)],
        axis=1)
    zeros = jnp.zeros((NPX, OUT_DIM), jnp.float32)
    parts = _sc_aggregate(xq, xkv, pk, dst, zeros)
    den = parts[:, NP:, :].reshape(NC, NP, DEN_W)
    return _finalize(parts, den, x, Wo_w,
                     Wo_b.reshape(1, -1), ln_g.reshape(1, -1),
                     ln_b.reshape(1, -1))


# speedup vs baseline: 1.3820x; 1.1641x over previous
import functools
import math

import jax
import jax.numpy as jnp
import numpy as np
from jax import lax
from jax.experimental import pallas as pl
from jax.experimental.pallas import tpu as pltpu
from jax.experimental.pallas import tpu_sc as plsc

N = 10000
E = 320000
IN_DIM = 128
OUT_DIM = 128
N_HEADS = 4
HEAD_DIM = OUT_DIM // N_HEADS
EDGE_DIM = 16
INV_SCALE = 1.0 / math.sqrt(HEAD_DIM)

NC = 2
NS = 16
NW = NC * NS
EW = E // NW
C = 40
CP = 48
CQ = C + 16
PKW = 224
NG = EW // C
NP = 10240
NPD = NP // 16
NPX = NP + NPD
RPT = NPX // NS

DEN_W = 8


def _proj_body(x_ref, wq_ref, wkv_ref, xq_ref, xkv_ref):
    x = x_ref[...]
    dn = (((1,), (1,)), ((), ()))
    xq_ref[...] = lax.dot_general(x, wq_ref[...], dn,
                                  preferred_element_type=jnp.float32)
    xkv_ref[...] = lax.dot_general(x, wkv_ref[...], dn,
                                   preferred_element_type=jnp.float32)


def _project(x, wq, wkv):
    bn = 2000
    grid = N // bn
    return pl.pallas_call(
        _proj_body,
        grid=(grid,),
        in_specs=[
            pl.BlockSpec((bn, IN_DIM), lambda i: (i, 0)),
            pl.BlockSpec((OUT_DIM, IN_DIM), lambda i: (0, 0)),
            pl.BlockSpec((2 * OUT_DIM, IN_DIM), lambda i: (0, 0)),
        ],
        out_specs=[
            pl.BlockSpec((bn, OUT_DIM), lambda i: (i, 0)),
            pl.BlockSpec((bn, 2 * OUT_DIM), lambda i: (i, 0)),
        ],
        out_shape=[
            jax.ShapeDtypeStruct((N, OUT_DIM), jnp.float32),
            jax.ShapeDtypeStruct((N, 2 * OUT_DIM), jnp.float32),
        ],
    )(x, wq, wkv)


def _bias_body(ea_ref, w1_ref, b1_ref, w2_ref, b2_ref, out_ref):
    dn = (((1,), (1,)), ((), ()))
    z = lax.dot_general(ea_ref[...], w1_ref[...], dn,
                        preferred_element_type=jnp.float32) + b1_ref[...]
    h = z * jax.nn.sigmoid(z)
    out_ref[...] = lax.dot_general(h, w2_ref[...], dn,
                                   preferred_element_type=jnp.float32) + b2_ref[...]


def _edge_bias(edge_attr, w1, b1, w2, b2):
    be = 4000
    grid = E // be
    return pl.pallas_call(
        _bias_body,
        grid=(grid,),
        in_specs=[
            pl.BlockSpec((be, EDGE_DIM), lambda i: (i, 0)),
            pl.BlockSpec((OUT_DIM, EDGE_DIM), lambda i: (0, 0)),
            pl.BlockSpec((1, OUT_DIM), lambda i: (0, 0)),
            pl.BlockSpec((N_HEADS, OUT_DIM), lambda i: (0, 0)),
            pl.BlockSpec((1, N_HEADS), lambda i: (0, 0)),
        ],
        out_specs=pl.BlockSpec((be, N_HEADS), lambda i: (i, 0)),
        out_shape=jax.ShapeDtypeStruct((E, N_HEADS), jnp.float32),
    )(edge_attr, w1, b1, w2, b2)


_GDN = lax.GatherDimensionNumbers(offset_dims=(), collapsed_slice_dims=(0,),
                                  start_index_map=(0,))


def _permute(vec, idx):
    return lax.gather(vec, idx[:, None], _GDN, (1,),
                      mode=lax.GatherScatterMode.PROMISE_IN_BOUNDS)


def _sc_body(xq_hbm, xkv_hbm, src_hbm, dst_hbm, bias_hbm,
             out_hbm,
             accum,
             srcb0, srcb1, srcb2, srcb3, srcb4,
             didxp0, didxp1, didxp2, didxp3, didxp4,
             biasb0, biasb1, biasb2, biasb3, biasb4,
             didxs0, didxs1, didx20, didx21,
             qbuf0, qbuf1, kvbuf0, kvbuf1, rowbuf, rowbuf2,
             gsem0, gsem1, ssem, isem0, isem1):
    srcb = (srcb0, srcb1, srcb2, srcb3, srcb4)
    didxp = (didxp0, didxp1, didxp2, didxp3, didxp4)
    biasb = (biasb0, biasb1, biasb2, biasb3, biasb4)
    didxs = (didxs0, didxs1)
    didx2 = (didx20, didx21)
    qbuf = (qbuf0, qbuf1)
    kvbuf = (kvbuf0, kvbuf1)
    gsem = (gsem0, gsem1)
    isem = (isem0, isem1)
    c = lax.axis_index("c")
    s = lax.axis_index("s")
    wid = s * NC + c
    ebase = wid * EW

    zvec = jnp.zeros((16,), jnp.float32)

    for i in range(C):
        for k in range(8):
            rowbuf2[i, pl.ds(16 * k, 16)] = zvec
    for t in range(RPT // C):
        pltpu.sync_copy(rowbuf2, accum.at[pl.ds(s * RPT + t * C, C)])
    for b in range(5):
        didxp[b][pl.ds(C, 16)] = jnp.zeros((16,), jnp.int32)
    plsc.subcore_barrier()

    lane = lax.iota(jnp.int32, 16)
    m8 = lane < 8

    def start_idx(b5, g, sem):
        base = ebase + g * C
        pltpu.async_copy(src_hbm.at[pl.ds(base, C)], srcb[b5], sem)
        pltpu.async_copy(dst_hbm.at[pl.ds(base, C)],
                         didxp[b5].at[pl.ds(0, C)], sem)
        pltpu.async_copy(bias_hbm.at[pl.ds(base * N_HEADS, C * N_HEADS)],
                         biasb[b5].at[pl.ds(0, C * N_HEADS)], sem)

    def drain_idx(b5, g, sem):
        base = ebase + g * C
        pltpu.make_async_copy(src_hbm.at[pl.ds(base, C)], srcb[b5],
                              sem).wait()
        pltpu.make_async_copy(dst_hbm.at[pl.ds(base, C)],
                              didxp[b5].at[pl.ds(0, C)], sem).wait()
        pltpu.make_async_copy(bias_hbm.at[pl.ds(base * N_HEADS,
                                                C * N_HEADS)],
                              biasb[b5].at[pl.ds(0, C * N_HEADS)],
                              sem).wait()

    def start_gather(b2, b5):
        sem = gsem[b2]
        pltpu.async_copy(xq_hbm.at[didxp[b5].at[pl.ds(0, C)]], qbuf[b2], sem)
        pltpu.async_copy(xkv_hbm.at[srcb[b5]], kvbuf[b2], sem)

    def drain_gather(b2, b5):
        sem = gsem[b2]
        pltpu.make_async_copy(xq_hbm.at[didxp[b5].at[pl.ds(0, C)]], qbuf[b2],
                              sem).wait()
        pltpu.make_async_copy(xkv_hbm.at[srcb[b5]], kvbuf[b2], sem).wait()

    def start_scatter(b2):
        pltpu.async_copy(rowbuf, accum.at[didxs[b2]], ssem, add=True)
        pltpu.async_copy(rowbuf2, accum.at[didx2[b2]], ssem, add=True)

    def drain_scatter(b2):
        pltpu.make_async_copy(rowbuf, accum.at[didxs[b2]], ssem).wait()
        pltpu.make_async_copy(rowbuf2, accum.at[didx2[b2]], ssem).wait()

    def compute(b2, b5):
        for j in range(CP // 16):
            dv = didxp[b5][pl.ds(16 * j, 16)]
            d2 = lax.shift_right_logical(dv, 4) + NP
            if 16 * (j + 1) <= C:
                didxs[b2][pl.ds(16 * j, 16)] = dv
                didx2[b2][pl.ds(16 * j, 16)] = d2
            else:
                plsc.store_scatter(didxs[b2], [lane + 16 * j], dv, mask=m8)
                plsc.store_scatter(didx2[b2], [lane + 16 * j], d2, mask=m8)

        @plsc.parallel_loop(0, C, 1, unroll=2)
        def edge(i):
            bvec = biasb[b5][pl.ds(i * N_HEADS, 16)]
            ex = []
            for h in range(N_HEADS):
                q0 = qbuf[b2][i, pl.ds(32 * h, 16)]
                q1 = qbuf[b2][i, pl.ds(32 * h + 16, 16)]
                k0 = kvbuf[b2][i, pl.ds(32 * h, 16)]
                k1 = kvbuf[b2][i, pl.ds(32 * h + 16, 16)]
                cs = plsc.cumsum(q0 * k0 + q1 * k1)
                logit = cs[15] * INV_SCALE + bvec[h]
                e_h = jnp.exp(jnp.full((16,), logit, jnp.float32))
                ex.append(e_h)
                v0 = kvbuf[b2][i, pl.ds(128 + 32 * h, 16)]
                v1 = kvbuf[b2][i, pl.ds(128 + 32 * h + 16, 16)]
                rowbuf[i, pl.ds(32 * h, 16)] = v0 * e_h
                rowbuf[i, pl.ds(32 * h + 16, 16)] = v1 * e_h
            den = jnp.where(
                lane == 0, ex[0],
                jnp.where(lane == 1, ex[1],
                          jnp.where(lane == 2, ex[2],
                                    jnp.where(lane == 3, ex[3], 1.0))))
            dvec = didxp[b5][pl.ds(i, 16)]
            pos = dvec[0] & 15
            sh8 = (pos & 1) * 8
            perm = (lane - sh8) & 15
            den_m = jnp.where(perm < 5, _permute(den, perm), 0.0)
            grp = lax.shift_right_logical(pos, 1)
            for k in range(8):
                rowbuf2[i, pl.ds(16 * k, 16)] = jnp.where(grp == k, den_m,
                                                          zvec)

    start_idx(0, 0, isem[0])
    start_idx(1, 1, isem[1])
    drain_idx(0, 0, isem[0])
    start_gather(0, 0)

    def outer(t, carry):
        for p in range(10):
            g = 10 * t + p
            b2 = p % 2
            b5 = p % 5

            @pl.when(g > 0)
            def _():
                drain_scatter(1 - b2)

            @pl.when(g + 2 < NG)
            def _():
                start_idx((p + 2) % 5, g + 2, isem[b2])

            @pl.when(g + 1 < NG)
            def _():
                drain_idx((p + 1) % 5, g + 1, isem[1 - b2])
                start_gather(1 - b2, (p + 1) % 5)

            drain_gather(b2, b5)
            compute(b2, b5)
            start_scatter(b2)
        return carry

    lax.fori_loop(0, NG // 10, outer, 0)
    drain_scatter(1)

    plsc.subcore_barrier()
    pltpu.sync_copy(accum.at[pl.ds(s * RPT, RPT)],
                    out_hbm.at[c, pl.ds(s * RPT, RPT)])


def _sc_aggregate(xq, xkv, src, dst, bias):
    mesh = plsc.VectorSubcoreMesh(core_axis_name="c", subcore_axis_name="s")
    fn = pl.kernel(
        _sc_body,
        out_type=jax.ShapeDtypeStruct((NC, NPX, OUT_DIM), jnp.float32),
        mesh=mesh,
        scratch_types=(
            [pltpu.VMEM_SHARED((NPX, OUT_DIM), jnp.float32)]
            + [pltpu.VMEM((C,), jnp.int32) for _ in range(5)]
            + [pltpu.VMEM((CQ,), jnp.int32) for _ in range(5)]
            + [pltpu.VMEM((C * N_HEADS + 16,), jnp.float32) for _ in range(5)]
            + [pltpu.VMEM((C,), jnp.int32) for _ in range(4)]
            + [pltpu.VMEM((C, OUT_DIM), jnp.float32) for _ in range(2)]
            + [pltpu.VMEM((C, 2 * OUT_DIM), jnp.float32) for _ in range(2)]
            + [pltpu.VMEM((C, OUT_DIM), jnp.float32) for _ in range(2)]
            + [pltpu.SemaphoreType.DMA for _ in range(5)]
        ),
        compiler_params=pltpu.CompilerParams(needs_layout_passes=False),
    )
    return fn(xq, xkv, src, dst, bias)


_EXPAND = np.kron(np.eye(N_HEADS, dtype=np.float32),
                  np.ones((1, HEAD_DIM), dtype=np.float32))


def _final_body(num_ref, den_ref, x_ref, wo_ref, wob_ref, g_ref, b_ref,
                exp_ref, out_ref):
    num = num_ref[0] + num_ref[1]
    dacc = den_ref[0] + den_ref[1]
    den4 = dacc[:, :N_HEADS]
    deg = dacc[:, N_HEADS:N_HEADS + 1]
    den = jnp.dot(den4, exp_ref[...], preferred_element_type=jnp.float32)
    aggr = num / (den + 1e-16)
    dn = (((1,), (1,)), ((), ()))
    msg = (lax.dot_general(aggr, wo_ref[...], dn,
                           preferred_element_type=jnp.float32)
           + deg * wob_ref[...])
    ge = 0.5 * msg * (1.0 + lax.erf(msg * (1.0 / math.sqrt(2.0))))
    y = x_ref[...] + ge
    mu = jnp.mean(y, axis=-1, keepdims=True)
    var = jnp.mean((y - mu) ** 2, axis=-1, keepdims=True)
    out_ref[...] = (y - mu) * lax.rsqrt(var + 1e-5) * g_ref[...] + b_ref[...]


def _finalize(num, den, x, wo, wob, ln_g, ln_b):
    bn = 2000
    grid = N // bn
    return pl.pallas_call(
        _final_body,
        grid=(grid,),
        in_specs=[
            pl.BlockSpec((NC, bn, OUT_DIM), lambda i: (0, i, 0)),
            pl.BlockSpec((NC, bn, DEN_W), lambda i: (0, i, 0)),
            pl.BlockSpec((bn, OUT_DIM), lambda i: (i, 0)),
            pl.BlockSpec((OUT_DIM, OUT_DIM), lambda i: (0, 0)),
            pl.BlockSpec((1, OUT_DIM), lambda i: (0, 0)),
            pl.BlockSpec((1, OUT_DIM), lambda i: (0, 0)),
            pl.BlockSpec((1, OUT_DIM), lambda i: (0, 0)),
            pl.BlockSpec((N_HEADS, OUT_DIM), lambda i: (0, 0)),
        ],
        out_specs=pl.BlockSpec((bn, OUT_DIM), lambda i: (i, 0)),
        out_shape=jax.ShapeDtypeStruct((N, OUT_DIM), jnp.float32),
    )(num, den, x, wo, wob, ln_g, ln_b, jnp.asarray(_EXPAND))


def kernel(x, edge_index, edge_attr, Wq, Wk, Wv, Ep1_w, Ep1_b, Ep2_w, Ep2_b,
           Wo_w, Wo_b, ln_g, ln_b):
    wkv = jnp.concatenate([Wk, Wv], axis=0)
    xq, xkv = _project(x, Wq, wkv)
    bias = _edge_bias(edge_attr, Ep1_w, Ep1_b.reshape(1, -1),
                      Ep2_w, Ep2_b.reshape(1, -1))
    src = edge_index[0]
    dst = edge_index[1]
    parts = _sc_aggregate(xq, xkv, src, dst, bias.reshape(-1))
    den = parts[:, NP:, :].reshape(NC, NP, DEN_W)
    return _finalize(parts, den, x, Wo_w,
                     Wo_b.reshape(1, -1), ln_g.reshape(1, -1),
                     ln_b.reshape(1, -1))


# speedup vs baseline: 1.4227x; 1.0294x over previous
import functools
import math

import jax
import jax.numpy as jnp
import numpy as np
from jax import lax
from jax.experimental import pallas as pl
from jax.experimental.pallas import tpu as pltpu
from jax.experimental.pallas import tpu_sc as plsc

N = 10000
E = 320000
IN_DIM = 128
OUT_DIM = 128
N_HEADS = 4
HEAD_DIM = OUT_DIM // N_HEADS
EDGE_DIM = 16
INV_SCALE = 1.0 / math.sqrt(HEAD_DIM)

NC = 2
NS = 16
NW = NC * NS
EW = E // NW
C = 40
CP = 48
CQ = C + 16
PKW = 224
NG = EW // C
NP = 10240
NPD = NP // 16
NPX = NP + NPD
RPT = NPX // NS

DEN_W = 8


def _pro_body(x_ref, ea_ref, wq_ref, wkv_ref, w1_ref, b1_ref, w2_ref, b2_ref,
              xq_ref, xkv_ref, bias_ref):
    dn = (((1,), (1,)), ((), ()))
    x = x_ref[...]
    xq_ref[...] = lax.dot_general(x, wq_ref[...], dn,
                                  preferred_element_type=jnp.float32)
    xkv_ref[...] = lax.dot_general(x, wkv_ref[...], dn,
                                   preferred_element_type=jnp.float32)
    z = lax.dot_general(ea_ref[...], w1_ref[...], dn,
                        preferred_element_type=jnp.float32) + b1_ref[...]
    h = z * jax.nn.sigmoid(z)
    bias_ref[...] = lax.dot_general(h, w2_ref[...], dn,
                                    preferred_element_type=jnp.float32) + b2_ref[...]


def _prologue(x, edge_attr, wq, wkv, w1, b1, w2, b2):
    grid = 50
    bn = N // grid
    be = E // grid
    return pl.pallas_call(
        _pro_body,
        grid=(grid,),
        in_specs=[
            pl.BlockSpec((bn, IN_DIM), lambda i: (i, 0)),
            pl.BlockSpec((be, EDGE_DIM), lambda i: (i, 0)),
            pl.BlockSpec((OUT_DIM, IN_DIM), lambda i: (0, 0)),
            pl.BlockSpec((2 * OUT_DIM, IN_DIM), lambda i: (0, 0)),
            pl.BlockSpec((OUT_DIM, EDGE_DIM), lambda i: (0, 0)),
            pl.BlockSpec((1, OUT_DIM), lambda i: (0, 0)),
            pl.BlockSpec((N_HEADS, OUT_DIM), lambda i: (0, 0)),
            pl.BlockSpec((1, N_HEADS), lambda i: (0, 0)),
        ],
        out_specs=[
            pl.BlockSpec((bn, OUT_DIM), lambda i: (i, 0)),
            pl.BlockSpec((bn, 2 * OUT_DIM), lambda i: (i, 0)),
            pl.BlockSpec((be, N_HEADS), lambda i: (i, 0)),
        ],
        out_shape=[
            jax.ShapeDtypeStruct((N, OUT_DIM), jnp.float32),
            jax.ShapeDtypeStruct((N, 2 * OUT_DIM), jnp.float32),
            jax.ShapeDtypeStruct((E, N_HEADS), jnp.float32),
        ],
    )(x, edge_attr, wq, wkv, w1, b1, w2, b2)


_GDN = lax.GatherDimensionNumbers(offset_dims=(), collapsed_slice_dims=(0,),
                                  start_index_map=(0,))


def _permute(vec, idx):
    return lax.gather(vec, idx[:, None], _GDN, (1,),
                      mode=lax.GatherScatterMode.PROMISE_IN_BOUNDS)


def _sc_body(xq_hbm, xkv_hbm, src_hbm, dst_hbm, bias_hbm,
             out_hbm,
             accum,
             srcb0, srcb1, srcb2, srcb3, srcb4,
             didxp0, didxp1, didxp2, didxp3, didxp4,
             biasb0, biasb1, biasb2, biasb3, biasb4,
             didxs0, didxs1, didx20, didx21,
             qbuf0, qbuf1, kvbuf0, kvbuf1, rowbuf, rowbuf2,
             gsem0, gsem1, ssem, isem0, isem1):
    srcb = (srcb0, srcb1, srcb2, srcb3, srcb4)
    didxp = (didxp0, didxp1, didxp2, didxp3, didxp4)
    biasb = (biasb0, biasb1, biasb2, biasb3, biasb4)
    didxs = (didxs0, didxs1)
    didx2 = (didx20, didx21)
    qbuf = (qbuf0, qbuf1)
    kvbuf = (kvbuf0, kvbuf1)
    gsem = (gsem0, gsem1)
    isem = (isem0, isem1)
    c = lax.axis_index("c")
    s = lax.axis_index("s")
    wid = s * NC + c
    ebase = wid * EW

    zvec = jnp.zeros((16,), jnp.float32)

    for i in range(C):
        for k in range(8):
            rowbuf2[i, pl.ds(16 * k, 16)] = zvec
    for t in range(RPT // C):
        pltpu.sync_copy(rowbuf2, accum.at[pl.ds(s * RPT + t * C, C)])
    for b in range(5):
        didxp[b][pl.ds(C, 16)] = jnp.zeros((16,), jnp.int32)
    plsc.subcore_barrier()

    lane = lax.iota(jnp.int32, 16)
    m8 = lane < 8

    def start_idx(b5, g, sem):
        base = ebase + g * C
        pltpu.async_copy(src_hbm.at[pl.ds(base, C)], srcb[b5], sem)
        pltpu.async_copy(dst_hbm.at[pl.ds(base, C)],
                         didxp[b5].at[pl.ds(0, C)], sem)
        pltpu.async_copy(bias_hbm.at[pl.ds(base * N_HEADS, C * N_HEADS)],
                         biasb[b5].at[pl.ds(0, C * N_HEADS)], sem)

    def drain_idx(b5, g, sem):
        base = ebase + g * C
        pltpu.make_async_copy(src_hbm.at[pl.ds(base, C)], srcb[b5],
                              sem).wait()
        pltpu.make_async_copy(dst_hbm.at[pl.ds(base, C)],
                              didxp[b5].at[pl.ds(0, C)], sem).wait()
        pltpu.make_async_copy(bias_hbm.at[pl.ds(base * N_HEADS,
                                                C * N_HEADS)],
                              biasb[b5].at[pl.ds(0, C * N_HEADS)],
                              sem).wait()

    def start_gather(b2, b5):
        sem = gsem[b2]
        pltpu.async_copy(xq_hbm.at[didxp[b5].at[pl.ds(0, C)]], qbuf[b2], sem)
        pltpu.async_copy(xkv_hbm.at[srcb[b5]], kvbuf[b2], sem)

    def drain_gather(b2, b5):
        sem = gsem[b2]
        pltpu.make_async_copy(xq_hbm.at[didxp[b5].at[pl.ds(0, C)]], qbuf[b2],
                              sem).wait()
        pltpu.make_async_copy(xkv_hbm.at[srcb[b5]], kvbuf[b2], sem).wait()

    def start_scatter(b2):
        pltpu.async_copy(rowbuf, accum.at[didxs[b2]], ssem, add=True)
        pltpu.async_copy(rowbuf2, accum.at[didx2[b2]], ssem, add=True)

    def drain_scatter(b2):
        pltpu.make_async_copy(rowbuf, accum.at[didxs[b2]], ssem).wait()
        pltpu.make_async_copy(rowbuf2, accum.at[didx2[b2]], ssem).wait()

    def compute(b2, b5):
        for j in range(CP // 16):
            dv = didxp[b5][pl.ds(16 * j, 16)]
            d2 = lax.shift_right_logical(dv, 4) + NP
            if 16 * (j + 1) <= C:
                didxs[b2][pl.ds(16 * j, 16)] = dv
                didx2[b2][pl.ds(16 * j, 16)] = d2
            else:
                plsc.store_scatter(didxs[b2], [lane + 16 * j], dv, mask=m8)
                plsc.store_scatter(didx2[b2], [lane + 16 * j], d2, mask=m8)

        @plsc.parallel_loop(0, C, 1, unroll=2)
        def edge(i):
            bvec = biasb[b5][pl.ds(i * N_HEADS, 16)]
            ex = []
            for h in range(N_HEADS):
                q0 = qbuf[b2][i, pl.ds(32 * h, 16)]
                q1 = qbuf[b2][i, pl.ds(32 * h + 16, 16)]
                k0 = kvbuf[b2][i, pl.ds(32 * h, 16)]
                k1 = kvbuf[b2][i, pl.ds(32 * h + 16, 16)]
                cs = plsc.cumsum(q0 * k0 + q1 * k1)
                logit = cs[15] * INV_SCALE + bvec[h]
                e_h = jnp.exp(jnp.full((16,), logit, jnp.float32))
                ex.append(e_h)
                v0 = kvbuf[b2][i, pl.ds(128 + 32 * h, 16)]
                v1 = kvbuf[b2][i, pl.ds(128 + 32 * h + 16, 16)]
                rowbuf[i, pl.ds(32 * h, 16)] = v0 * e_h
                rowbuf[i, pl.ds(32 * h + 16, 16)] = v1 * e_h
            den = jnp.where(
                lane == 0, ex[0],
                jnp.where(lane == 1, ex[1],
                          jnp.where(lane == 2, ex[2],
                                    jnp.where(lane == 3, ex[3], 1.0))))
            dvec = didxp[b5][pl.ds(i, 16)]
            pos = dvec[0] & 15
            sh8 = (pos & 1) * 8
            perm = (lane - sh8) & 15
            den_m = jnp.where(perm < 5, _permute(den, perm), 0.0)
            grp = lax.shift_right_logical(pos, 1)
            for k in range(8):
                rowbuf2[i, pl.ds(16 * k, 16)] = jnp.where(grp == k, den_m,
                                                          zvec)

    start_idx(0, 0, isem[0])
    start_idx(1, 1, isem[1])
    drain_idx(0, 0, isem[0])
    start_gather(0, 0)

    def outer(t, carry):
        for p in range(10):
            g = 10 * t + p
            b2 = p % 2
            b5 = p % 5

            @pl.when(g > 0)
            def _():
                drain_scatter(1 - b2)

            @pl.when(g + 2 < NG)
            def _():
                start_idx((p + 2) % 5, g + 2, isem[b2])

            @pl.when(g + 1 < NG)
            def _():
                drain_idx((p + 1) % 5, g + 1, isem[1 - b2])
                start_gather(1 - b2, (p + 1) % 5)

            drain_gather(b2, b5)
            compute(b2, b5)
            start_scatter(b2)
        return carry

    lax.fori_loop(0, NG // 10, outer, 0)
    drain_scatter(1)

    plsc.subcore_barrier()
    pltpu.sync_copy(accum.at[pl.ds(s * RPT, RPT)],
                    out_hbm.at[c, pl.ds(s * RPT, RPT)])


def _sc_aggregate(xq, xkv, src, dst, bias):
    mesh = plsc.VectorSubcoreMesh(core_axis_name="c", subcore_axis_name="s")
    fn = pl.kernel(
        _sc_body,
        out_type=jax.ShapeDtypeStruct((NC, NPX, OUT_DIM), jnp.float32),
        mesh=mesh,
        scratch_types=(
            [pltpu.VMEM_SHARED((NPX, OUT_DIM), jnp.float32)]
            + [pltpu.VMEM((C,), jnp.int32) for _ in range(5)]
            + [pltpu.VMEM((CQ,), jnp.int32) for _ in range(5)]
            + [pltpu.VMEM((C * N_HEADS + 16,), jnp.float32) for _ in range(5)]
            + [pltpu.VMEM((C,), jnp.int32) for _ in range(4)]
            + [pltpu.VMEM((C, OUT_DIM), jnp.float32) for _ in range(2)]
            + [pltpu.VMEM((C, 2 * OUT_DIM), jnp.float32) for _ in range(2)]
            + [pltpu.VMEM((C, OUT_DIM), jnp.float32) for _ in range(2)]
            + [pltpu.SemaphoreType.DMA for _ in range(5)]
        ),
        compiler_params=pltpu.CompilerParams(needs_layout_passes=False),
    )
    return fn(xq, xkv, src, dst, bias)


_EXPAND = np.kron(np.eye(N_HEADS, dtype=np.float32),
                  np.ones((1, HEAD_DIM), dtype=np.float32))


def _final_body(num_ref, den_ref, x_ref, wo_ref, wob_ref, g_ref, b_ref,
                exp_ref, out_ref):
    num = num_ref[0] + num_ref[1]
    dacc = den_ref[0] + den_ref[1]
    den4 = dacc[:, :N_HEADS]
    deg = dacc[:, N_HEADS:N_HEADS + 1]
    den = jnp.dot(den4, exp_ref[...], preferred_element_type=jnp.float32)
    aggr = num / (den + 1e-16)
    dn = (((1,), (1,)), ((), ()))
    msg = (lax.dot_general(aggr, wo_ref[...], dn,
                           preferred_element_type=jnp.float32)
           + deg * wob_ref[...])
    ge = 0.5 * msg * (1.0 + lax.erf(msg * (1.0 / math.sqrt(2.0))))
    y = x_ref[...] + ge
    mu = jnp.mean(y, axis=-1, keepdims=True)
    var = jnp.mean((y - mu) ** 2, axis=-1, keepdims=True)
    out_ref[...] = (y - mu) * lax.rsqrt(var + 1e-5) * g_ref[...] + b_ref[...]


def _finalize(num, den, x, wo, wob, ln_g, ln_b):
    bn = 2000
    grid = N // bn
    return pl.pallas_call(
        _final_body,
        grid=(grid,),
        in_specs=[
            pl.BlockSpec((NC, bn, OUT_DIM), lambda i: (0, i, 0)),
            pl.BlockSpec((NC, bn, DEN_W), lambda i: (0, i, 0)),
            pl.BlockSpec((bn, OUT_DIM), lambda i: (i, 0)),
            pl.BlockSpec((OUT_DIM, OUT_DIM), lambda i: (0, 0)),
            pl.BlockSpec((1, OUT_DIM), lambda i: (0, 0)),
            pl.BlockSpec((1, OUT_DIM), lambda i: (0, 0)),
            pl.BlockSpec((1, OUT_DIM), lambda i: (0, 0)),
            pl.BlockSpec((N_HEADS, OUT_DIM), lambda i: (0, 0)),
        ],
        out_specs=pl.BlockSpec((bn, OUT_DIM), lambda i: (i, 0)),
        out_shape=jax.ShapeDtypeStruct((N, OUT_DIM), jnp.float32),
    )(num, den, x, wo, wob, ln_g, ln_b, jnp.asarray(_EXPAND))


def kernel(x, edge_index, edge_attr, Wq, Wk, Wv, Ep1_w, Ep1_b, Ep2_w, Ep2_b,
           Wo_w, Wo_b, ln_g, ln_b):
    wkv = jnp.concatenate([Wk, Wv], axis=0)
    xq, xkv, bias = _prologue(x, edge_attr, Wq, wkv, Ep1_w,
                              Ep1_b.reshape(1, -1), Ep2_w,
                              Ep2_b.reshape(1, -1))
    src = edge_index[0]
    dst = edge_index[1]
    parts = _sc_aggregate(xq, xkv, src, dst, bias.reshape(-1))
    den = parts[:, NP:, :].reshape(NC, NP, DEN_W)
    return _finalize(parts, den, x, Wo_w,
                     Wo_b.reshape(1, -1), ln_g.reshape(1, -1),
                     ln_b.reshape(1, -1))


# speedup vs baseline: 1.4322x; 1.0067x over previous
import functools
import math

import jax
import jax.numpy as jnp
import numpy as np
from jax import lax
from jax.experimental import pallas as pl
from jax.experimental.pallas import tpu as pltpu
from jax.experimental.pallas import tpu_sc as plsc

N = 10000
E = 320000
IN_DIM = 128
OUT_DIM = 128
N_HEADS = 4
HEAD_DIM = OUT_DIM // N_HEADS
EDGE_DIM = 16
INV_SCALE = 1.0 / math.sqrt(HEAD_DIM)

NC = 2
NS = 16
NW = NC * NS
EW = E // NW
C = 40
CP = 48
CQ = C + 16
PKW = 224
NG = EW // C
NP = 10240
NPD = NP // 16
NPX = NP + NPD
RPT = NPX // NS

DEN_W = 8


def _pro_body(x_ref, ea_ref, wq_ref, wk_ref, wv_ref, w1_ref, b1_ref,
              w2_ref, b2_ref, xq_ref, xkv_ref, bias_ref):
    dn = (((1,), (1,)), ((), ()))
    x = x_ref[...]
    xq_ref[...] = lax.dot_general(x, wq_ref[...], dn,
                                  preferred_element_type=jnp.float32)
    xkv_ref[:, :OUT_DIM] = lax.dot_general(x, wk_ref[...], dn,
                                           preferred_element_type=jnp.float32)
    xkv_ref[:, OUT_DIM:] = lax.dot_general(x, wv_ref[...], dn,
                                           preferred_element_type=jnp.float32)
    z = lax.dot_general(ea_ref[...], w1_ref[...], dn,
                        preferred_element_type=jnp.float32) + b1_ref[...]
    h = z * jax.nn.sigmoid(z)
    bias_ref[...] = lax.dot_general(h, w2_ref[...], dn,
                                    preferred_element_type=jnp.float32) + b2_ref[...]


def _prologue(x, edge_attr, wq, wk, wv, w1, b1, w2, b2):
    grid = 50
    bn = N // grid
    be = E // grid
    return pl.pallas_call(
        _pro_body,
        grid=(grid,),
        in_specs=[
            pl.BlockSpec((bn, IN_DIM), lambda i: (i, 0)),
            pl.BlockSpec((be, EDGE_DIM), lambda i: (i, 0)),
            pl.BlockSpec((OUT_DIM, IN_DIM), lambda i: (0, 0)),
            pl.BlockSpec((OUT_DIM, IN_DIM), lambda i: (0, 0)),
            pl.BlockSpec((OUT_DIM, IN_DIM), lambda i: (0, 0)),
            pl.BlockSpec((OUT_DIM, EDGE_DIM), lambda i: (0, 0)),
            pl.BlockSpec((1, OUT_DIM), lambda i: (0, 0)),
            pl.BlockSpec((N_HEADS, OUT_DIM), lambda i: (0, 0)),
            pl.BlockSpec((1, N_HEADS), lambda i: (0, 0)),
        ],
        out_specs=[
            pl.BlockSpec((bn, OUT_DIM), lambda i: (i, 0)),
            pl.BlockSpec((bn, 2 * OUT_DIM), lambda i: (i, 0)),
            pl.BlockSpec((be, N_HEADS), lambda i: (i, 0)),
        ],
        out_shape=[
            jax.ShapeDtypeStruct((N, OUT_DIM), jnp.float32),
            jax.ShapeDtypeStruct((N, 2 * OUT_DIM), jnp.float32),
            jax.ShapeDtypeStruct((E, N_HEADS), jnp.float32),
        ],
    )(x, edge_attr, wq, wk, wv, w1, b1, w2, b2)


_GDN = lax.GatherDimensionNumbers(offset_dims=(), collapsed_slice_dims=(0,),
                                  start_index_map=(0,))


def _permute(vec, idx):
    return lax.gather(vec, idx[:, None], _GDN, (1,),
                      mode=lax.GatherScatterMode.PROMISE_IN_BOUNDS)


def _sc_body(xq_hbm, xkv_hbm, src_hbm, dst_hbm, bias_hbm,
             out_hbm,
             accum,
             srcb0, srcb1, srcb2, srcb3, srcb4,
             didxp0, didxp1, didxp2, didxp3, didxp4,
             biasb0, biasb1, biasb2, biasb3, biasb4,
             didxs0, didxs1, didx20, didx21,
             qbuf0, qbuf1, kvbuf0, kvbuf1, rowbuf, rowbuf2,
             gsem0, gsem1, ssem, isem0, isem1):
    srcb = (srcb0, srcb1, srcb2, srcb3, srcb4)
    didxp = (didxp0, didxp1, didxp2, didxp3, didxp4)
    biasb = (biasb0, biasb1, biasb2, biasb3, biasb4)
    didxs = (didxs0, didxs1)
    didx2 = (didx20, didx21)
    qbuf = (qbuf0, qbuf1)
    kvbuf = (kvbuf0, kvbuf1)
    gsem = (gsem0, gsem1)
    isem = (isem0, isem1)
    c = lax.axis_index("c")
    s = lax.axis_index("s")
    wid = s * NC + c
    ebase = wid * EW

    zvec = jnp.zeros((16,), jnp.float32)

    for i in range(C):
        for k in range(8):
            rowbuf2[i, pl.ds(16 * k, 16)] = zvec
    for t in range(RPT // C):
        pltpu.sync_copy(rowbuf2, accum.at[pl.ds(s * RPT + t * C, C)])
    for b in range(5):
        didxp[b][pl.ds(C, 16)] = jnp.zeros((16,), jnp.int32)
    plsc.subcore_barrier()

    lane = lax.iota(jnp.int32, 16)
    m8 = lane < 8

    def start_idx(b5, g, sem):
        base = ebase + g * C
        pltpu.async_copy(src_hbm.at[pl.ds(base, C)], srcb[b5], sem)
        pltpu.async_copy(dst_hbm.at[pl.ds(base, C)],
                         didxp[b5].at[pl.ds(0, C)], sem)
        pltpu.async_copy(bias_hbm.at[pl.ds(base * N_HEADS, C * N_HEADS)],
                         biasb[b5].at[pl.ds(0, C * N_HEADS)], sem)

    def drain_idx(b5, g, sem):
        base = ebase + g * C
        pltpu.make_async_copy(src_hbm.at[pl.ds(base, C)], srcb[b5],
                              sem).wait()
        pltpu.make_async_copy(dst_hbm.at[pl.ds(base, C)],
                              didxp[b5].at[pl.ds(0, C)], sem).wait()
        pltpu.make_async_copy(bias_hbm.at[pl.ds(base * N_HEADS,
                                                C * N_HEADS)],
                              biasb[b5].at[pl.ds(0, C * N_HEADS)],
                              sem).wait()

    def start_gather(b2, b5):
        sem = gsem[b2]
        pltpu.async_copy(xq_hbm.at[didxp[b5].at[pl.ds(0, C)]], qbuf[b2], sem)
        pltpu.async_copy(xkv_hbm.at[srcb[b5]], kvbuf[b2], sem)

    def drain_gather(b2, b5):
        sem = gsem[b2]
        pltpu.make_async_copy(xq_hbm.at[didxp[b5].at[pl.ds(0, C)]], qbuf[b2],
                              sem).wait()
        pltpu.make_async_copy(xkv_hbm.at[srcb[b5]], kvbuf[b2], sem).wait()

    def start_scatter(b2):
        pltpu.async_copy(rowbuf, accum.at[didxs[b2]], ssem, add=True)
        pltpu.async_copy(rowbuf2, accum.at[didx2[b2]], ssem, add=True)

    def drain_scatter(b2):
        pltpu.make_async_copy(rowbuf, accum.at[didxs[b2]], ssem).wait()
        pltpu.make_async_copy(rowbuf2, accum.at[didx2[b2]], ssem).wait()

    def compute(b2, b5):
        for j in range(CP // 16):
            dv = didxp[b5][pl.ds(16 * j, 16)]
            d2 = lax.shift_right_logical(dv, 4) + NP
            if 16 * (j + 1) <= C:
                didxs[b2][pl.ds(16 * j, 16)] = dv
                didx2[b2][pl.ds(16 * j, 16)] = d2
            else:
                plsc.store_scatter(didxs[b2], [lane + 16 * j], dv, mask=m8)
                plsc.store_scatter(didx2[b2], [lane + 16 * j], d2, mask=m8)

        @plsc.parallel_loop(0, C, 1, unroll=2)
        def edge(i):
            bvec = biasb[b5][pl.ds(i * N_HEADS, 16)]
            ex = []
            for h in range(N_HEADS):
                q0 = qbuf[b2][i, pl.ds(32 * h, 16)]
                q1 = qbuf[b2][i, pl.ds(32 * h + 16, 16)]
                k0 = kvbuf[b2][i, pl.ds(32 * h, 16)]
                k1 = kvbuf[b2][i, pl.ds(32 * h + 16, 16)]
                cs = plsc.cumsum(q0 * k0 + q1 * k1)
                logit = cs[15] * INV_SCALE + bvec[h]
                e_h = jnp.exp(jnp.full((16,), logit, jnp.float32))
                ex.append(e_h)
                v0 = kvbuf[b2][i, pl.ds(128 + 32 * h, 16)]
                v1 = kvbuf[b2][i, pl.ds(128 + 32 * h + 16, 16)]
                rowbuf[i, pl.ds(32 * h, 16)] = v0 * e_h
                rowbuf[i, pl.ds(32 * h + 16, 16)] = v1 * e_h
            den = jnp.where(
                lane == 0, ex[0],
                jnp.where(lane == 1, ex[1],
                          jnp.where(lane == 2, ex[2],
                                    jnp.where(lane == 3, ex[3], 1.0))))
            dvec = didxp[b5][pl.ds(i, 16)]
            pos = dvec[0] & 15
            sh8 = (pos & 1) * 8
            perm = (lane - sh8) & 15
            den_m = jnp.where(perm < 5, _permute(den, perm), 0.0)
            grp = lax.shift_right_logical(pos, 1)
            for k in range(8):
                rowbuf2[i, pl.ds(16 * k, 16)] = jnp.where(grp == k, den_m,
                                                          zvec)

    start_idx(0, 0, isem[0])
    start_idx(1, 1, isem[1])
    drain_idx(0, 0, isem[0])
    start_gather(0, 0)

    def outer(t, carry):
        for p in range(10):
            g = 10 * t + p
            b2 = p % 2
            b5 = p % 5

            @pl.when(g > 0)
            def _():
                drain_scatter(1 - b2)

            @pl.when(g + 2 < NG)
            def _():
                start_idx((p + 2) % 5, g + 2, isem[b2])

            @pl.when(g + 1 < NG)
            def _():
                drain_idx((p + 1) % 5, g + 1, isem[1 - b2])
                start_gather(1 - b2, (p + 1) % 5)

            drain_gather(b2, b5)
            compute(b2, b5)
            start_scatter(b2)
        return carry

    lax.fori_loop(0, NG // 10, outer, 0)
    drain_scatter(1)

    plsc.subcore_barrier()
    pltpu.sync_copy(accum.at[pl.ds(s * RPT, RPT)],
                    out_hbm.at[c, pl.ds(s * RPT, RPT)])


def _sc_aggregate(xq, xkv, src, dst, bias):
    mesh = plsc.VectorSubcoreMesh(core_axis_name="c", subcore_axis_name="s")
    fn = pl.kernel(
        _sc_body,
        out_type=jax.ShapeDtypeStruct((NC, NPX, OUT_DIM), jnp.float32),
        mesh=mesh,
        scratch_types=(
            [pltpu.VMEM_SHARED((NPX, OUT_DIM), jnp.float32)]
            + [pltpu.VMEM((C,), jnp.int32) for _ in range(5)]
            + [pltpu.VMEM((CQ,), jnp.int32) for _ in range(5)]
            + [pltpu.VMEM((C * N_HEADS + 16,), jnp.float32) for _ in range(5)]
            + [pltpu.VMEM((C,), jnp.int32) for _ in range(4)]
            + [pltpu.VMEM((C, OUT_DIM), jnp.float32) for _ in range(2)]
            + [pltpu.VMEM((C, 2 * OUT_DIM), jnp.float32) for _ in range(2)]
            + [pltpu.VMEM((C, OUT_DIM), jnp.float32) for _ in range(2)]
            + [pltpu.SemaphoreType.DMA for _ in range(5)]
        ),
        compiler_params=pltpu.CompilerParams(needs_layout_passes=False),
    )
    return fn(xq, xkv, src, dst, bias)


_EXPAND = np.kron(np.eye(N_HEADS, dtype=np.float32),
                  np.ones((1, HEAD_DIM), dtype=np.float32))


def _final_body(num_ref, den_ref, x_ref, wo_ref, wob_ref, g_ref, b_ref,
                exp_ref, out_ref):
    num = num_ref[0] + num_ref[1]
    dacc = den_ref[0] + den_ref[1]
    den4 = dacc[:, :N_HEADS]
    deg = dacc[:, N_HEADS:N_HEADS + 1]
    den = jnp.dot(den4, exp_ref[...], preferred_element_type=jnp.float32)
    aggr = num / (den + 1e-16)
    dn = (((1,), (1,)), ((), ()))
    msg = (lax.dot_general(aggr, wo_ref[...], dn,
                           preferred_element_type=jnp.float32)
           + deg * wob_ref[...])
    ge = 0.5 * msg * (1.0 + lax.erf(msg * (1.0 / math.sqrt(2.0))))
    y = x_ref[...] + ge
    mu = jnp.mean(y, axis=-1, keepdims=True)
    var = jnp.mean((y - mu) ** 2, axis=-1, keepdims=True)
    out_ref[...] = (y - mu) * lax.rsqrt(var + 1e-5) * g_ref[...] + b_ref[...]


def _finalize(num, den, x, wo, wob, ln_g, ln_b):
    bn = 2000
    grid = N // bn
    return pl.pallas_call(
        _final_body,
        grid=(grid,),
        in_specs=[
            pl.BlockSpec((NC, bn, OUT_DIM), lambda i: (0, i, 0)),
            pl.BlockSpec((NC, bn, DEN_W), lambda i: (0, i, 0)),
            pl.BlockSpec((bn, OUT_DIM), lambda i: (i, 0)),
            pl.BlockSpec((OUT_DIM, OUT_DIM), lambda i: (0, 0)),
            pl.BlockSpec((1, OUT_DIM), lambda i: (0, 0)),
            pl.BlockSpec((1, OUT_DIM), lambda i: (0, 0)),
            pl.BlockSpec((1, OUT_DIM), lambda i: (0, 0)),
            pl.BlockSpec((N_HEADS, OUT_DIM), lambda i: (0, 0)),
        ],
        out_specs=pl.BlockSpec((bn, OUT_DIM), lambda i: (i, 0)),
        out_shape=jax.ShapeDtypeStruct((N, OUT_DIM), jnp.float32),
    )(num, den, x, wo, wob, ln_g, ln_b, jnp.asarray(_EXPAND))


def kernel(x, edge_index, edge_attr, Wq, Wk, Wv, Ep1_w, Ep1_b, Ep2_w, Ep2_b,
           Wo_w, Wo_b, ln_g, ln_b):
    xq, xkv, bias = _prologue(x, edge_attr, Wq, Wk, Wv, Ep1_w,
                              Ep1_b.reshape(1, -1), Ep2_w,
                              Ep2_b.reshape(1, -1))
    parts = _sc_aggregate(xq, xkv, edge_index[0], edge_index[1],
                          bias.reshape(-1))
    den = parts[:, NP:, :].reshape(NC, NP, DEN_W)
    return _finalize(parts, den, x, Wo_w,
                     Wo_b.reshape(1, -1), ln_g.reshape(1, -1),
                     ln_b.reshape(1, -1))


# speedup vs baseline: 1.5113x; 1.0552x over previous
import functools
import math

import jax
import jax.numpy as jnp
import numpy as np
from jax import lax
from jax.experimental import pallas as pl
from jax.experimental.pallas import tpu as pltpu
from jax.experimental.pallas import tpu_sc as plsc

N = 10000
E = 320000
IN_DIM = 128
OUT_DIM = 128
N_HEADS = 4
HEAD_DIM = OUT_DIM // N_HEADS
EDGE_DIM = 16
INV_SCALE = 1.0 / math.sqrt(HEAD_DIM)

NC = 2
NS = 16
NW = NC * NS
EW = E // NW
C = 40
CP = 48
CQ = C + 16
PKW = 224
NG = EW // C
NP = 10240
NPD = NP // 16
NPX = NP + NPD
RPT = NPX // NS

DEN_W = 8


def _pro_body(x_ref, ea_ref, wq_ref, wk_ref, wv_ref, w1_ref, b1_ref,
              w2_ref, b2_ref, xq_ref, xkv_ref, bias_ref):
    dn = (((1,), (1,)), ((), ()))
    x = x_ref[...]
    xq_ref[...] = lax.dot_general(x, wq_ref[...], dn,
                                  preferred_element_type=jnp.float32)
    xkv_ref[:, :OUT_DIM] = lax.dot_general(x, wk_ref[...], dn,
                                           preferred_element_type=jnp.float32)
    xkv_ref[:, OUT_DIM:] = lax.dot_general(x, wv_ref[...], dn,
                                           preferred_element_type=jnp.float32)
    z = lax.dot_general(ea_ref[...], w1_ref[...], dn,
                        preferred_element_type=jnp.float32) + b1_ref[...]
    h = z * jax.nn.sigmoid(z)
    bias_ref[...] = lax.dot_general(h, w2_ref[...], dn,
                                    preferred_element_type=jnp.float32) + b2_ref[...]


def _prologue(x, edge_attr, wq, wk, wv, w1, b1, w2, b2):
    grid = 50
    bn = N // grid
    be = E // grid
    return pl.pallas_call(
        _pro_body,
        grid=(grid,),
        in_specs=[
            pl.BlockSpec((bn, IN_DIM), lambda i: (i, 0)),
            pl.BlockSpec((be, EDGE_DIM), lambda i: (i, 0)),
            pl.BlockSpec((OUT_DIM, IN_DIM), lambda i: (0, 0)),
            pl.BlockSpec((OUT_DIM, IN_DIM), lambda i: (0, 0)),
            pl.BlockSpec((OUT_DIM, IN_DIM), lambda i: (0, 0)),
            pl.BlockSpec((OUT_DIM, EDGE_DIM), lambda i: (0, 0)),
            pl.BlockSpec((1, OUT_DIM), lambda i: (0, 0)),
            pl.BlockSpec((N_HEADS, OUT_DIM), lambda i: (0, 0)),
            pl.BlockSpec((1, N_HEADS), lambda i: (0, 0)),
        ],
        out_specs=[
            pl.BlockSpec((bn, OUT_DIM), lambda i: (i, 0)),
            pl.BlockSpec((bn, 2 * OUT_DIM), lambda i: (i, 0)),
            pl.BlockSpec((be, N_HEADS), lambda i: (i, 0)),
        ],
        out_shape=[
            jax.ShapeDtypeStruct((N, OUT_DIM), jnp.float32),
            jax.ShapeDtypeStruct((N, 2 * OUT_DIM), jnp.float32),
            jax.ShapeDtypeStruct((E, N_HEADS), jnp.float32),
        ],
    )(x, edge_attr, wq, wk, wv, w1, b1, w2, b2)


_GDN = lax.GatherDimensionNumbers(offset_dims=(), collapsed_slice_dims=(0,),
                                  start_index_map=(0,))


def _permute(vec, idx):
    return lax.gather(vec, idx[:, None], _GDN, (1,),
                      mode=lax.GatherScatterMode.PROMISE_IN_BOUNDS)


def _sc_body(xq_hbm, xkv_hbm, src_hbm, dst_hbm, bias_hbm,
             out_hbm,
             accum,
             srcb0, srcb1,
             didxp0, didxp1,
             biasb0, biasb1,
             didxs0, didxs1, didx20, didx21,
             qbuf0, qbuf1, kvbuf0, kvbuf1, rowbuf, rowbuf2,
             gsem0, gsem1, ssem, isem0, isem1):
    srcb = (srcb0, srcb1)
    didxp = (didxp0, didxp1)
    biasb = (biasb0, biasb1)
    didxs = (didxs0, didxs1)
    didx2 = (didx20, didx21)
    qbuf = (qbuf0, qbuf1)
    kvbuf = (kvbuf0, kvbuf1)
    gsem = (gsem0, gsem1)
    isem = (isem0, isem1)
    c = lax.axis_index("c")
    s = lax.axis_index("s")
    wid = s * NC + c
    ebase = wid * EW

    zvec = jnp.zeros((16,), jnp.float32)

    for i in range(C):
        for k in range(8):
            rowbuf2[i, pl.ds(16 * k, 16)] = zvec
    for t in range(RPT // C):
        pltpu.sync_copy(rowbuf2, accum.at[pl.ds(s * RPT + t * C, C)])
    for b in range(2):
        didxp[b][pl.ds(C, 16)] = jnp.zeros((16,), jnp.int32)
    plsc.subcore_barrier()

    lane = lax.iota(jnp.int32, 16)
    m8 = lane < 8

    def start_idx(b5, g, sem):
        base = ebase + g * C
        pltpu.async_copy(src_hbm.at[pl.ds(base, C)], srcb[b5], sem)
        pltpu.async_copy(dst_hbm.at[pl.ds(base, C)],
                         didxp[b5].at[pl.ds(0, C)], sem)
        pltpu.async_copy(bias_hbm.at[pl.ds(base * N_HEADS, C * N_HEADS)],
                         biasb[b5].at[pl.ds(0, C * N_HEADS)], sem)

    def drain_idx(b5, g, sem):
        base = ebase + g * C
        pltpu.make_async_copy(src_hbm.at[pl.ds(base, C)], srcb[b5],
                              sem).wait()
        pltpu.make_async_copy(dst_hbm.at[pl.ds(base, C)],
                              didxp[b5].at[pl.ds(0, C)], sem).wait()
        pltpu.make_async_copy(bias_hbm.at[pl.ds(base * N_HEADS,
                                                C * N_HEADS)],
                              biasb[b5].at[pl.ds(0, C * N_HEADS)],
                              sem).wait()

    def start_gather(b2, b5):
        sem = gsem[b2]
        pltpu.async_copy(xq_hbm.at[didxp[b5].at[pl.ds(0, C)]], qbuf[b2], sem)
        pltpu.async_copy(xkv_hbm.at[srcb[b5]], kvbuf[b2], sem)

    def drain_gather(b2, b5):
        sem = gsem[b2]
        pltpu.make_async_copy(xq_hbm.at[didxp[b5].at[pl.ds(0, C)]], qbuf[b2],
                              sem).wait()
        pltpu.make_async_copy(xkv_hbm.at[srcb[b5]], kvbuf[b2], sem).wait()

    def start_scatter(b2):
        pltpu.async_copy(rowbuf, accum.at[didxs[b2]], ssem, add=True)
        pltpu.async_copy(rowbuf2, accum.at[didx2[b2]], ssem, add=True)

    def drain_scatter(b2):
        pltpu.make_async_copy(rowbuf, accum.at[didxs[b2]], ssem).wait()
        pltpu.make_async_copy(rowbuf2, accum.at[didx2[b2]], ssem).wait()

    def compute(b2, b5):
        for j in range(CP // 16):
            dv = didxp[b5][pl.ds(16 * j, 16)]
            d2 = lax.shift_right_logical(dv, 4) + NP
            if 16 * (j + 1) <= C:
                didxs[b2][pl.ds(16 * j, 16)] = dv
                didx2[b2][pl.ds(16 * j, 16)] = d2
            else:
                plsc.store_scatter(didxs[b2], [lane + 16 * j], dv, mask=m8)
                plsc.store_scatter(didx2[b2], [lane + 16 * j], d2, mask=m8)

        @plsc.parallel_loop(0, C, 1, unroll=2)
        def edge(i):
            bvec = biasb[b5][pl.ds(i * N_HEADS, 16)]
            ex = []
            for h in range(N_HEADS):
                q0 = qbuf[b2][i, pl.ds(32 * h, 16)]
                q1 = qbuf[b2][i, pl.ds(32 * h + 16, 16)]
                k0 = kvbuf[b2][i, pl.ds(32 * h, 16)]
                k1 = kvbuf[b2][i, pl.ds(32 * h + 16, 16)]
                cs = plsc.cumsum(q0 * k0 + q1 * k1)
                logit = cs[15] * INV_SCALE + bvec[h]
                e_h = jnp.exp(jnp.full((16,), logit, jnp.float32))
                ex.append(e_h)
                v0 = kvbuf[b2][i, pl.ds(128 + 32 * h, 16)]
                v1 = kvbuf[b2][i, pl.ds(128 + 32 * h + 16, 16)]
                rowbuf[i, pl.ds(32 * h, 16)] = v0 * e_h
                rowbuf[i, pl.ds(32 * h + 16, 16)] = v1 * e_h
            den = jnp.where(
                lane == 0, ex[0],
                jnp.where(lane == 1, ex[1],
                          jnp.where(lane == 2, ex[2],
                                    jnp.where(lane == 3, ex[3], 1.0))))
            dvec = didxp[b5][pl.ds(i, 16)]
            pos = dvec[0] & 15
            sh8 = (pos & 1) * 8
            perm = (lane - sh8) & 15
            den_m = jnp.where(perm < 5, _permute(den, perm), 0.0)
            grp = lax.shift_right_logical(pos, 1)
            for k in range(8):
                rowbuf2[i, pl.ds(16 * k, 16)] = jnp.where(grp == k, den_m,
                                                          zvec)

    start_idx(0, 0, isem[0])
    start_idx(1, 1, isem[1])
    drain_idx(0, 0, isem[0])
    start_gather(0, 0)

    def outer(t, carry):
        for b in range(2):
            g = 2 * t + b
            nb = 1 - b

            @pl.when(g > 0)
            def _():
                drain_scatter(nb)

            @pl.when(g + 1 < NG)
            def _():
                drain_idx(nb, g + 1, isem[nb])
                start_gather(nb, nb)

            drain_gather(b, b)
            compute(b, b)
            start_scatter(b)

            @pl.when(g + 2 < NG)
            def _():
                start_idx(b, g + 2, isem[b])
        return carry

    lax.fori_loop(0, NG // 2, outer, 0)
    drain_scatter(1)

    plsc.subcore_barrier()
    pltpu.sync_copy(accum.at[pl.ds(s * RPT, RPT)],
                    out_hbm.at[c, pl.ds(s * RPT, RPT)])


def _sc_aggregate(xq, xkv, src, dst, bias):
    mesh = plsc.VectorSubcoreMesh(core_axis_name="c", subcore_axis_name="s")
    fn = pl.kernel(
        _sc_body,
        out_type=jax.ShapeDtypeStruct((NC, NPX, OUT_DIM), jnp.float32),
        mesh=mesh,
        scratch_types=(
            [pltpu.VMEM_SHARED((NPX, OUT_DIM), jnp.float32)]
            + [pltpu.VMEM((C,), jnp.int32) for _ in range(2)]
            + [pltpu.VMEM((CQ,), jnp.int32) for _ in range(2)]
            + [pltpu.VMEM((C * N_HEADS + 16,), jnp.float32) for _ in range(2)]
            + [pltpu.VMEM((C,), jnp.int32) for _ in range(4)]
            + [pltpu.VMEM((C, OUT_DIM), jnp.float32) for _ in range(2)]
            + [pltpu.VMEM((C, 2 * OUT_DIM), jnp.float32) for _ in range(2)]
            + [pltpu.VMEM((C, OUT_DIM), jnp.float32) for _ in range(2)]
            + [pltpu.SemaphoreType.DMA for _ in range(5)]
        ),
        compiler_params=pltpu.CompilerParams(needs_layout_passes=False),
    )
    return fn(xq, xkv, src, dst, bias)


_EXPAND = np.kron(np.eye(N_HEADS, dtype=np.float32),
                  np.ones((1, HEAD_DIM), dtype=np.float32))


def _final_body(num_ref, den_ref, x_ref, wo_ref, wob_ref, g_ref, b_ref,
                exp_ref, out_ref):
    num = num_ref[0] + num_ref[1]
    dacc = den_ref[0] + den_ref[1]
    den4 = dacc[:, :N_HEADS]
    deg = dacc[:, N_HEADS:N_HEADS + 1]
    den = jnp.dot(den4, exp_ref[...], preferred_element_type=jnp.float32)
    aggr = num / (den + 1e-16)
    dn = (((1,), (1,)), ((), ()))
    msg = (lax.dot_general(aggr, wo_ref[...], dn,
                           preferred_element_type=jnp.float32)
           + deg * wob_ref[...])
    ge = 0.5 * msg * (1.0 + lax.erf(msg * (1.0 / math.sqrt(2.0))))
    y = x_ref[...] + ge
    mu = jnp.mean(y, axis=-1, keepdims=True)
    var = jnp.mean((y - mu) ** 2, axis=-1, keepdims=True)
    out_ref[...] = (y - mu) * lax.rsqrt(var + 1e-5) * g_ref[...] + b_ref[...]


def _finalize(num, den, x, wo, wob, ln_g, ln_b):
    bn = 2000
    grid = N // bn
    return pl.pallas_call(
        _final_body,
        grid=(grid,),
        in_specs=[
            pl.BlockSpec((NC, bn, OUT_DIM), lambda i: (0, i, 0)),
            pl.BlockSpec((NC, bn, DEN_W), lambda i: (0, i, 0)),
            pl.BlockSpec((bn, OUT_DIM), lambda i: (i, 0)),
            pl.BlockSpec((OUT_DIM, OUT_DIM), lambda i: (0, 0)),
            pl.BlockSpec((1, OUT_DIM), lambda i: (0, 0)),
            pl.BlockSpec((1, OUT_DIM), lambda i: (0, 0)),
            pl.BlockSpec((1, OUT_DIM), lambda i: (0, 0)),
            pl.BlockSpec((N_HEADS, OUT_DIM), lambda i: (0, 0)),
        ],
        out_specs=pl.BlockSpec((bn, OUT_DIM), lambda i: (i, 0)),
        out_shape=jax.ShapeDtypeStruct((N, OUT_DIM), jnp.float32),
    )(num, den, x, wo, wob, ln_g, ln_b, jnp.asarray(_EXPAND))


def kernel(x, edge_index, edge_attr, Wq, Wk, Wv, Ep1_w, Ep1_b, Ep2_w, Ep2_b,
           Wo_w, Wo_b, ln_g, ln_b):
    xq, xkv, bias = _prologue(x, edge_attr, Wq, Wk, Wv, Ep1_w,
                              Ep1_b.reshape(1, -1), Ep2_w,
                              Ep2_b.reshape(1, -1))
    parts = _sc_aggregate(xq, xkv, edge_index[0], edge_index[1],
                          bias.reshape(-1))
    den = parts[:, NP:, :].reshape(NC, NP, DEN_W)
    return _finalize(parts, den, x, Wo_w,
                     Wo_b.reshape(1, -1), ln_g.reshape(1, -1),
                     ln_b.reshape(1, -1))
